# Initial kernel scaffold; baseline (speedup 1.0000x reference)
import jax
import jax.numpy as jnp
from jax.experimental import pallas as pl


def kernel(x, pe, edge_attr, params, edge_index, batch):
    raise NotImplementedError("write your pallas kernel here")


# speedup vs baseline: 22.8790x; 22.8790x over previous
import functools

import jax
import jax.numpy as jnp
from jax import lax
from jax.experimental import pallas as pl
from jax.experimental.pallas import tpu as pltpu
from jax.experimental.pallas import tpu_sc as plsc

N = 50000
E = 800000
G = 1000
C = 64
H = 4
DH = 16
NL = 3

TQ = 400
TK = 400
NQT = N // TQ

BT = 1000
NMT = N // BT

ER = E // 128
ERP = 6256
ERB = ERP // 8

NSUB = 16
NP = 50048
NZR = NP // NSUB
WB = 184
NWB = NZR // WB

NEG = -1e9
F32 = jnp.float32


def _eidx_body(a0, a1, a2, a3, s_ref, gid_ref):
    best = a0[...]
    bi = jnp.zeros(best.shape, jnp.int32)
    for j, ar in enumerate((a1, a2, a3), start=1):
        v = ar[...]
        bi = jnp.where(v > best, j, bi)
        best = jnp.maximum(best, v)
    gid_ref[...] = bi * N + s_ref[...]


def _emit_pre(h, wi_ref, bi_ref, emb_ref, q_ref, k_ref, v_ref, ra_ref, rb_ref):
    qkv = jnp.dot(h, wi_ref[...], preferred_element_type=F32) + bi_ref[...]
    q_ref[...] = qkv[:, 0:C]
    k_ref[...] = qkv[:, C:2 * C]
    v_ref[...] = qkv[:, 2 * C:3 * C]
    for cc in range(4):
        rc = jnp.maximum(h + emb_ref[cc:cc + 1, :], 0.0)
        ra_ref[cc, :, :] = rc[:, :32]
        rb_ref[cc, :, :] = rc[:, 32:]


def _k0_body(x_ref, w0_ref, b0_ref, wi_ref, bi_ref, emb_ref,
             h_ref, q_ref, k_ref, v_ref, ra_ref, rb_ref):
    h = jnp.dot(x_ref[...], w0_ref[...], preferred_element_type=F32) + b0_ref[...]
    h_ref[...] = h
    _emit_pre(h, wi_ref, bi_ref, emb_ref, q_ref, k_ref, v_ref, ra_ref, rb_ref)


def _mid_common(h_ref, agg_ref, h2_ref, w1, b1, w2, b2, mw1, mb1, mw2, mb2,
                n1g, n1b, n3g, n3b):
    h = h_ref[...]
    ag = agg_ref[...]
    agg = jnp.concatenate([ag[0], ag[1]], axis=1)
    t = h + agg
    t = jnp.dot(jnp.maximum(jnp.dot(t, w1[...], preferred_element_type=F32)
                            + b1[...], 0.0),
                w2[...], preferred_element_type=F32) + b2[...]
    h1 = (t + h) * n1g[...] + n1b[...]
    out = h1 + h2_ref[...]
    mlp = jnp.dot(jnp.maximum(jnp.dot(out, mw1[...], preferred_element_type=F32)
                              + mb1[...], 0.0),
                  mw2[...], preferred_element_type=F32) + mb2[...]
    return (out + mlp) * n3g[...] + n3b[...]


def _mid_body(h_ref, agg_ref, h2_ref, w1, b1, w2, b2, mw1, mb1, mw2, mb2,
              n1g, n1b, n3g, n3b, wi_ref, bi_ref, emb_ref,
              hn_ref, q_ref, k_ref, v_ref, ra_ref, rb_ref):
    hn = _mid_common(h_ref, agg_ref, h2_ref, w1, b1, w2, b2,
                     mw1, mb1, mw2, mb2, n1g, n1b, n3g, n3b)
    hn_ref[...] = hn
    _emit_pre(hn, wi_ref, bi_ref, emb_ref, q_ref, k_ref, v_ref, ra_ref, rb_ref)


def _pool_body(h_ref, agg_ref, h2_ref, w1, b1, w2, b2, mw1, mb1, mw2, mb2,
               n1g, n1b, n3g, n3b, bat_ref, pool_ref):
    hn = _mid_common(h_ref, agg_ref, h2_ref, w1, b1, w2, b2,
                     mw1, mb1, mw2, mb2, n1g, n1b, n3g, n3b)
    bt = bat_ref[0]
    oh = (lax.broadcasted_iota(jnp.int32, (G, BT), 0) == bt).astype(F32)
    contrib = jnp.dot(oh, hn, preferred_element_type=F32)

    @pl.when(pl.program_id(0) == 0)
    def _init():
        pool_ref[...] = jnp.zeros_like(pool_ref)

    pool_ref[...] = pool_ref[...] + contrib


def _attn_body(ks_ref, nk_ref, q_ref, qlo_ref, qhi_ref, h_ref,
               wo_ref, bo_ref, g_ref, bb_ref, k_hbm, v_hbm,
               o_ref, kbuf, vbuf, sk, sv):
    i = pl.program_id(0)
    start = ks_ref[i]
    nk = nk_ref[i]
    q = q_ref[...]
    qlo = qlo_ref[...]
    qhi = qhi_ref[...]
    qs = [q[:, DH * hh:DH * (hh + 1)] * 0.25 for hh in range(H)]

    def body(kt, carry):
        ms, ls, accs = carry
        off = start + kt * TK
        ck = pltpu.make_async_copy(k_hbm.at[pl.ds(off, TK)], kbuf, sk)
        cv = pltpu.make_async_copy(v_hbm.at[pl.ds(off, TK)], vbuf, sv)
        ck.start()
        cv.start()
        ck.wait()
        cv.wait()
        kk = kbuf[...]
        vv = vbuf[...]
        col = off + lax.broadcasted_iota(jnp.int32, (TQ, TK), 1)
        valid = (col >= qlo) & (col < qhi)
        nms, nls, naccs = [], [], []
        for hh in range(H):
            sc = lax.dot_general(qs[hh], kk[:, DH * hh:DH * (hh + 1)],
                                 (((1,), (1,)), ((), ())),
                                 preferred_element_type=F32)
            sc = jnp.where(valid, sc, NEG)
            mn = jnp.maximum(ms[hh], jnp.max(sc, axis=1, keepdims=True))
            al = jnp.exp(ms[hh] - mn)
            ps = jnp.exp(sc - mn)
            nls.append(ls[hh] * al + jnp.sum(ps, axis=1, keepdims=True))
            pv = lax.dot_general(ps, vv[:, DH * hh:DH * (hh + 1)],
                                 (((1,), (0,)), ((), ())),
                                 preferred_element_type=F32)
            naccs.append(accs[hh] * al + pv)
            nms.append(mn)
        return tuple(nms), tuple(nls), tuple(naccs)

    m0 = tuple(jnp.full((TQ, 1), -1e30, F32) for _ in range(H))
    l0 = tuple(jnp.zeros((TQ, 1), F32) for _ in range(H))
    a0 = tuple(jnp.zeros((TQ, DH), F32) for _ in range(H))
    ms, ls, accs = lax.fori_loop(jnp.int32(0), nk, body, (m0, l0, a0))
    o = jnp.concatenate([accs[hh] / ls[hh] for hh in range(H)], axis=1)
    a_out = jnp.dot(o, wo_ref[...], preferred_element_type=F32) + bo_ref[...]
    o_ref[...] = (a_out + h_ref[...]) * g_ref[...] + bb_ref[...]


_sc_mesh = plsc.VectorSubcoreMesh(core_axis_name="c", subcore_axis_name="s")


@functools.partial(
    pl.kernel,
    mesh=_sc_mesh,
    compiler_params=pltpu.CompilerParams(use_tc_tiling_on_sc=False),
    out_type=jax.ShapeDtypeStruct((2 * NP, 32), F32),
    scratch_types=[
        pltpu.VMEM((8, 128), jnp.int32),
        pltpu.VMEM((8, 128), jnp.int32),
        pltpu.VMEM((128, 32), F32),
        pltpu.VMEM((WB, 32), F32),
        pltpu.VMEM_SHARED((NP, 32), F32),
        pltpu.SemaphoreType.DMA,
    ],
)
def _sc_edge_agg(gid_hbm, dst_hbm, r0_hbm, r1_hbm, zero_hbm, out_hbm,
                 gbuf, dbuf, rows, zbuf, aggsh, sem):
    c = lax.axis_index("c")
    s = lax.axis_index("s")

    pltpu.sync_copy(zero_hbm, zbuf)
    base = s * NZR
    for j in range(NWB):
        pltpu.sync_copy(zbuf, aggsh.at[pl.ds(base + j * WB, WB)])
    plsc.subcore_barrier()

    nblk = jnp.int32(ERB // NSUB) + (s < ERB % NSUB).astype(jnp.int32)

    def body(i, carry):
        b = s + i * NSUB
        pltpu.sync_copy(gid_hbm.at[pl.ds(b * 8, 8)], gbuf)
        pltpu.sync_copy(dst_hbm.at[pl.ds(b * 8, 8)], dbuf)
        for j in range(8):
            @pl.when(c == 0)
            def _g0():
                pltpu.async_copy(r0_hbm.at[gbuf.at[jnp.int32(j)]], rows, sem).wait()

            @pl.when(c == 1)
            def _g1():
                pltpu.async_copy(r1_hbm.at[gbuf.at[jnp.int32(j)]], rows, sem).wait()

            pltpu.sync_copy(rows, aggsh.at[dbuf.at[jnp.int32(j)]], add=True)
        return carry

    lax.fori_loop(jnp.int32(0), nblk, body, jnp.int32(0))
    plsc.subcore_barrier()

    for j in range(NWB):
        pltpu.sync_copy(aggsh.at[pl.ds(base + j * WB, WB)], zbuf)
        pltpu.sync_copy(zbuf, out_hbm.at[pl.ds(c * NP + base + j * WB, WB)])


def _Z():
    return jnp.int32(0)


def _const_spec(shape):
    return pl.BlockSpec(shape, lambda i: tuple(_Z() for _ in shape))


def kernel(x, pe, edge_attr, params, edge_index, batch):
    p = params
    batch32 = batch.astype(jnp.int32)
    src = edge_index[0].astype(jnp.int32)
    dst = edge_index[1].astype(jnp.int32)

    xf = x.astype(F32)
    X = jnp.concatenate([xf, pe.astype(F32), jnp.zeros((N, 3), F32)], axis=1)
    w0 = jnp.zeros((32, C), F32)
    w0 = w0.at[0:9, 0:C - 8].set(p["node_W"].T)
    pe_w = p["pe_W"] * p["pe_norm_g"][None, :]
    pe_b = p["pe_W"] @ p["pe_norm_b"] + p["pe_b"]
    w0 = w0.at[9:29, C - 8:C].set(pe_w.T)
    b0 = jnp.concatenate([p["node_b"], pe_b])[None, :]

    emb = p["edge_emb"].astype(F32)
    wiT = [p["attn_Wi"][l].T for l in range(NL)]
    biT = [p["attn_bi"][l][None, :] for l in range(NL)]
    woT = [p["attn_Wo"][l].T for l in range(NL)]
    boT = [p["attn_bo"][l][None, :] for l in range(NL)]
    w1T = [p["nn_W1"][l].T for l in range(NL)]
    b1T = [p["nn_b1"][l][None, :] for l in range(NL)]
    w2T = [p["nn_W2"][l].T for l in range(NL)]
    b2T = [p["nn_b2"][l][None, :] for l in range(NL)]
    mw1T = [p["mlp_W1"][l].T for l in range(NL)]
    mb1T = [p["mlp_b1"][l][None, :] for l in range(NL)]
    mw2T = [p["mlp_W2"][l].T for l in range(NL)]
    mb2T = [p["mlp_b2"][l][None, :] for l in range(NL)]
    ng = {nm: [p[nm + "_g"][l][None, :] for l in range(NL)] for nm in ("n1", "n2", "n3")}
    nb = {nm: [p[nm + "_b"][l][None, :] for l in range(NL)] for nm in ("n1", "n2", "n3")}

    pad_e = ERP * 128 - E

    def _erp(v, dt, fill=0):
        return jnp.pad(v.astype(dt), (0, pad_e),
                       constant_values=fill).reshape(ERP, 128)

    acols = [_erp(edge_attr[:, j], F32) for j in range(4)]
    src2 = _erp(src, jnp.int32)
    dst2 = _erp(dst, jnp.int32, fill=N)

    seg_off = jnp.searchsorted(batch32, jnp.arange(G + 1, dtype=jnp.int32)).astype(jnp.int32)
    qlo2 = seg_off[batch32][:, None]
    qhi2 = seg_off[batch32 + 1][:, None]
    g_first = batch32[::TQ]
    g_last = batch32[TQ - 1::TQ]
    kstart = (seg_off[g_first] // TK) * TK
    nkt = (seg_off[g_last + 1] - kstart + TK - 1) // TK
    bat3 = batch32.reshape(NMT, 1, BT)
    zero_wb = jnp.zeros((WB, 32), F32)

    gid2 = pl.pallas_call(
        _eidx_body,
        out_shape=jax.ShapeDtypeStruct((ERP, 128), jnp.int32),
    )(*acols, src2)

    row_spec = pl.BlockSpec((BT, C), lambda i: (i, _Z()))
    r_spec = pl.BlockSpec((4, BT, 32), lambda i: (_Z(), i, _Z()))
    nodes_f32 = jax.ShapeDtypeStruct((N, C), F32)
    rtab = jax.ShapeDtypeStruct((4, N, 32), F32)

    h, q, k, v, ra, rb = pl.pallas_call(
        _k0_body,
        grid=(NMT,),
        in_specs=[pl.BlockSpec((BT, 32), lambda i: (i, _Z())),
                  _const_spec((32, C)), _const_spec((1, C)),
                  _const_spec((C, 3 * C)), _const_spec((1, 3 * C)),
                  _const_spec((4, C))],
        out_specs=[row_spec] * 4 + [r_spec] * 2,
        out_shape=[nodes_f32] * 4 + [rtab] * 2,
    )(X, w0, b0, wiT[0], biT[0], emb)

    pooled = None
    for l in range(NL):
        agg2 = _sc_edge_agg(gid2, dst2, ra.reshape(4 * N, 32),
                            rb.reshape(4 * N, 32), zero_wb)
        agg3 = agg2.reshape(2, NP, 32)

        h2 = pl.pallas_call(
            _attn_body,
            grid=(NQT,),
            in_specs=[pl.BlockSpec((NQT,), lambda i: (_Z(),),
                                   memory_space=pltpu.SMEM),
                      pl.BlockSpec((NQT,), lambda i: (_Z(),),
                                   memory_space=pltpu.SMEM),
                      pl.BlockSpec((TQ, C), lambda i: (i, _Z())),
                      pl.BlockSpec((TQ, 1), lambda i: (i, _Z())),
                      pl.BlockSpec((TQ, 1), lambda i: (i, _Z())),
                      pl.BlockSpec((TQ, C), lambda i: (i, _Z())),
                      _const_spec((C, C)), _const_spec((1, C)),
                      _const_spec((1, C)), _const_spec((1, C)),
                      pl.BlockSpec(memory_space=pl.ANY),
                      pl.BlockSpec(memory_space=pl.ANY)],
            out_specs=pl.BlockSpec((TQ, C), lambda i: (i, _Z())),
            out_shape=nodes_f32,
            scratch_shapes=[pltpu.VMEM((TK, C), F32), pltpu.VMEM((TK, C), F32),
                            pltpu.SemaphoreType.DMA, pltpu.SemaphoreType.DMA],
        )(kstart, nkt, q, qlo2, qhi2, h, woT[l], boT[l],
          ng["n2"][l], nb["n2"][l], k, v)

        mid_w = (w1T[l], b1T[l], w2T[l], b2T[l],
                 mw1T[l], mb1T[l], mw2T[l], mb2T[l],
                 ng["n1"][l], nb["n1"][l], ng["n3"][l], nb["n3"][l])
        mid_w_specs = [_const_spec((C, C)), _const_spec((1, C)),
                       _const_spec((C, C)), _const_spec((1, C)),
                       _const_spec((C, 2 * C)), _const_spec((1, 2 * C)),
                       _const_spec((2 * C, C)), _const_spec((1, C)),
                       _const_spec((1, C)), _const_spec((1, C)),
                       _const_spec((1, C)), _const_spec((1, C))]
        agg_spec = pl.BlockSpec((2, BT, 32), lambda i: (_Z(), i, _Z()))

        if l < NL - 1:
            h, q, k, v, ra, rb = pl.pallas_call(
                _mid_body,
                grid=(NMT,),
                in_specs=[row_spec, agg_spec, row_spec] + mid_w_specs +
                         [_const_spec((C, 3 * C)), _const_spec((1, 3 * C)),
                          _const_spec((4, C))],
                out_specs=[row_spec] * 4 + [r_spec] * 2,
                out_shape=[nodes_f32] * 4 + [rtab] * 2,
            )(h, agg3, h2, *mid_w, wiT[l + 1], biT[l + 1], emb)
        else:
            pooled = pl.pallas_call(
                _pool_body,
                grid=(NMT,),
                in_specs=[row_spec, agg_spec, row_spec] + mid_w_specs +
                         [pl.BlockSpec((1, 1, BT), lambda i: (i, _Z(), _Z()))],
                out_specs=pl.BlockSpec((G, C), lambda i: (_Z(), _Z())),
                out_shape=jax.ShapeDtypeStruct((G, C), F32),
            )(h, agg3, h2, *mid_w, bat3)

    return pooled


# speedup vs baseline: 24.6817x; 1.0788x over previous
import functools

import jax
import jax.numpy as jnp
from jax import lax
from jax.experimental import pallas as pl
from jax.experimental.pallas import tpu as pltpu
from jax.experimental.pallas import tpu_sc as plsc

N = 50000
E = 800000
G = 1000
C = 64
H = 4
DH = 16
NL = 3

TQ = 400
TK = 400
NQT = N // TQ

BT = 1000
NMT = N // BT

ER = E // 128
ERP = 6256
EGB = 16
NGRP = ERP // EGB

NSUB = 16
NP = 50048
NZR = NP // NSUB
WB = 184
NWB = NZR // WB

NEG = -1e9
F32 = jnp.float32


def _eidx_body(a0, a1, a2, a3, s_ref, gid_ref):
    best = a0[...]
    bi = jnp.zeros(best.shape, jnp.int32)
    for j, ar in enumerate((a1, a2, a3), start=1):
        v = ar[...]
        bi = jnp.where(v > best, j, bi)
        best = jnp.maximum(best, v)
    gid_ref[...] = bi * N + s_ref[...]


def _emit_pre(h, wi_ref, bi_ref, emb_ref, q_ref, kv_ref, ra_ref, rb_ref):
    qkv = jnp.dot(h, wi_ref[...], preferred_element_type=F32) + bi_ref[...]
    q_ref[...] = qkv[:, 0:C]
    kv_ref[...] = qkv[:, C:3 * C]
    for cc in range(4):
        rc = jnp.maximum(h + emb_ref[cc:cc + 1, :], 0.0)
        ra_ref[cc, :, :] = rc[:, :32]
        rb_ref[cc, :, :] = rc[:, 32:]


def _k0_body(x_ref, w0_ref, b0_ref, wi_ref, bi_ref, emb_ref,
             h_ref, q_ref, kv_ref, ra_ref, rb_ref):
    h = jnp.dot(x_ref[...], w0_ref[...], preferred_element_type=F32) + b0_ref[...]
    h_ref[...] = h
    _emit_pre(h, wi_ref, bi_ref, emb_ref, q_ref, kv_ref, ra_ref, rb_ref)


def _mid_common(h_ref, agg_ref, h2_ref, w1, b1, w2, b2, mw1, mb1, mw2, mb2,
                n1g, n1b, n3g, n3b):
    h = h_ref[...]
    ag = agg_ref[...]
    agg = jnp.concatenate([ag[0], ag[1]], axis=1)
    t = h + agg
    t = jnp.dot(jnp.maximum(jnp.dot(t, w1[...], preferred_element_type=F32)
                            + b1[...], 0.0),
                w2[...], preferred_element_type=F32) + b2[...]
    h1 = (t + h) * n1g[...] + n1b[...]
    out = h1 + h2_ref[...]
    mlp = jnp.dot(jnp.maximum(jnp.dot(out, mw1[...], preferred_element_type=F32)
                              + mb1[...], 0.0),
                  mw2[...], preferred_element_type=F32) + mb2[...]
    return (out + mlp) * n3g[...] + n3b[...]


def _mid_body(h_ref, agg_ref, h2_ref, w1, b1, w2, b2, mw1, mb1, mw2, mb2,
              n1g, n1b, n3g, n3b, wi_ref, bi_ref, emb_ref,
              hn_ref, q_ref, kv_ref, ra_ref, rb_ref):
    hn = _mid_common(h_ref, agg_ref, h2_ref, w1, b1, w2, b2,
                     mw1, mb1, mw2, mb2, n1g, n1b, n3g, n3b)
    hn_ref[...] = hn
    _emit_pre(hn, wi_ref, bi_ref, emb_ref, q_ref, kv_ref, ra_ref, rb_ref)


def _pool_body(h_ref, agg_ref, h2_ref, w1, b1, w2, b2, mw1, mb1, mw2, mb2,
               n1g, n1b, n3g, n3b, bat_ref, pool_ref):
    hn = _mid_common(h_ref, agg_ref, h2_ref, w1, b1, w2, b2,
                     mw1, mb1, mw2, mb2, n1g, n1b, n3g, n3b)
    bt = bat_ref[0]
    oh = (lax.broadcasted_iota(jnp.int32, (G, BT), 0) == bt).astype(F32)
    contrib = jnp.dot(oh, hn, preferred_element_type=F32)

    @pl.when(pl.program_id(0) == 0)
    def _init():
        pool_ref[...] = jnp.zeros_like(pool_ref)

    pool_ref[...] = pool_ref[...] + contrib


def _attn_body(ks_ref, nk_ref, q_ref, qlo_ref, qhi_ref, h_ref,
               wo_ref, bo_ref, g_ref, bb_ref, kv_hbm,
               o_ref, kvbuf, sems):
    i = pl.program_id(0)
    start = ks_ref[i]
    nk = nk_ref[i]
    q = q_ref[...]
    qlo = qlo_ref[...]
    qhi = qhi_ref[...]
    qs = [q[:, DH * hh:DH * (hh + 1)] * 0.25 for hh in range(H)]

    def desc(kt, par):
        off = start + kt * TK
        return pltpu.make_async_copy(kv_hbm.at[pl.ds(off, TK)],
                                     kvbuf.at[par], sems.at[par])

    desc(jnp.int32(0), jnp.int32(0)).start()

    def body(kt, carry):
        ms, ls, accs = carry
        par = lax.rem(kt, jnp.int32(2))

        @pl.when(kt + 1 < nk)
        def _pf():
            desc(kt + 1, 1 - par).start()

        desc(kt, par).wait()
        kv = kvbuf[par]
        kk = kv[:, 0:C]
        vv = kv[:, C:2 * C]
        off = start + kt * TK
        col = off + lax.broadcasted_iota(jnp.int32, (TQ, TK), 1)
        valid = (col >= qlo) & (col < qhi)
        nms, nls, naccs = [], [], []
        for hh in range(H):
            sc = lax.dot_general(qs[hh], kk[:, DH * hh:DH * (hh + 1)],
                                 (((1,), (1,)), ((), ())),
                                 preferred_element_type=F32)
            sc = jnp.where(valid, sc, NEG)
            mn = jnp.maximum(ms[hh], jnp.max(sc, axis=1, keepdims=True))
            al = jnp.exp(ms[hh] - mn)
            ps = jnp.exp(sc - mn)
            nls.append(ls[hh] * al + jnp.sum(ps, axis=1, keepdims=True))
            pv = lax.dot_general(ps, vv[:, DH * hh:DH * (hh + 1)],
                                 (((1,), (0,)), ((), ())),
                                 preferred_element_type=F32)
            naccs.append(accs[hh] * al + pv)
            nms.append(mn)
        return tuple(nms), tuple(nls), tuple(naccs)

    m0 = tuple(jnp.full((TQ, 1), -1e30, F32) for _ in range(H))
    l0 = tuple(jnp.zeros((TQ, 1), F32) for _ in range(H))
    a0 = tuple(jnp.zeros((TQ, DH), F32) for _ in range(H))
    ms, ls, accs = lax.fori_loop(jnp.int32(0), nk, body, (m0, l0, a0))
    o = jnp.concatenate([accs[hh] / ls[hh] for hh in range(H)], axis=1)
    a_out = jnp.dot(o, wo_ref[...], preferred_element_type=F32) + bo_ref[...]
    o_ref[...] = (a_out + h_ref[...]) * g_ref[...] + bb_ref[...]


_sc_mesh = plsc.VectorSubcoreMesh(core_axis_name="c", subcore_axis_name="s")


@functools.partial(
    pl.kernel,
    mesh=_sc_mesh,
    compiler_params=pltpu.CompilerParams(use_tc_tiling_on_sc=False),
    out_type=jax.ShapeDtypeStruct((2 * NP, 32), F32),
    scratch_types=[
        pltpu.VMEM((EGB, 128), jnp.int32),
        pltpu.VMEM((EGB, 128), jnp.int32),
        pltpu.VMEM((2, 2, 128, 32), F32),
        pltpu.VMEM((WB, 32), F32),
        pltpu.VMEM_SHARED((NP, 32), F32),
        pltpu.SemaphoreType.DMA,
        pltpu.SemaphoreType.DMA,
    ],
)
def _sc_edge_agg(gid_hbm, dst_hbm, r0_hbm, r1_hbm, zero_hbm, out_hbm,
                 gbuf, dbuf, rows, zbuf, aggsh, gsem, ssem):
    c = lax.axis_index("c")
    s = lax.axis_index("s")

    pltpu.sync_copy(zero_hbm, zbuf)
    base = s * NZR
    for j in range(NWB):
        pltpu.sync_copy(zbuf, aggsh.at[pl.ds(base + j * WB, WB)])
    plsc.subcore_barrier()

    ngrp = jnp.int32(NGRP // NSUB) + (s < NGRP % NSUB).astype(jnp.int32)
    nsub = EGB // 2

    def body(i, carry):
        g = s + i * NSUB
        pltpu.sync_copy(gid_hbm.at[pl.ds(g * EGB, EGB)], gbuf)
        pltpu.sync_copy(dst_hbm.at[pl.ds(g * EGB, EGB)], dbuf)

        def pipe(r_hbm):
            def gfire(t):
                p = t % 2
                return [pltpu.async_copy(
                    r_hbm.at[gbuf.at[jnp.int32(2 * t + j)]],
                    rows.at[jnp.int32(p), jnp.int32(j)], gsem)
                    for j in range(2)]

            def sfire(t):
                p = t % 2
                return [pltpu.async_copy(
                    rows.at[jnp.int32(p), jnp.int32(j)],
                    aggsh.at[dbuf.at[jnp.int32(2 * t + j)]], ssem, add=True)
                    for j in range(2)]

            gd = gfire(0)
            sd_prev = None
            for t in range(nsub):
                for d in gd:
                    d.wait()
                sd = sfire(t)
                if sd_prev is not None:
                    for d in sd_prev:
                        d.wait()
                if t < nsub - 1:
                    gd = gfire(t + 1)
                sd_prev = sd
            for d in sd_prev:
                d.wait()

        @pl.when(c == 0)
        def _p0():
            pipe(r0_hbm)

        @pl.when(c == 1)
        def _p1():
            pipe(r1_hbm)

        return carry

    lax.fori_loop(jnp.int32(0), ngrp, body, jnp.int32(0))
    plsc.subcore_barrier()

    for j in range(NWB):
        pltpu.sync_copy(aggsh.at[pl.ds(base + j * WB, WB)], zbuf)
        pltpu.sync_copy(zbuf, out_hbm.at[pl.ds(c * NP + base + j * WB, WB)])


def _Z():
    return jnp.int32(0)


def _const_spec(shape):
    return pl.BlockSpec(shape, lambda i: tuple(_Z() for _ in shape))


def kernel(x, pe, edge_attr, params, edge_index, batch):
    p = params
    batch32 = batch.astype(jnp.int32)
    src = edge_index[0].astype(jnp.int32)
    dst = edge_index[1].astype(jnp.int32)

    xf = x.astype(F32)
    X = jnp.concatenate([xf, pe.astype(F32), jnp.zeros((N, 3), F32)], axis=1)
    w0 = jnp.zeros((32, C), F32)
    w0 = w0.at[0:9, 0:C - 8].set(p["node_W"].T)
    pe_w = p["pe_W"] * p["pe_norm_g"][None, :]
    pe_b = p["pe_W"] @ p["pe_norm_b"] + p["pe_b"]
    w0 = w0.at[9:29, C - 8:C].set(pe_w.T)
    b0 = jnp.concatenate([p["node_b"], pe_b])[None, :]

    emb = p["edge_emb"].astype(F32)
    wiT = [p["attn_Wi"][l].T for l in range(NL)]
    biT = [p["attn_bi"][l][None, :] for l in range(NL)]
    woT = [p["attn_Wo"][l].T for l in range(NL)]
    boT = [p["attn_bo"][l][None, :] for l in range(NL)]
    w1T = [p["nn_W1"][l].T for l in range(NL)]
    b1T = [p["nn_b1"][l][None, :] for l in range(NL)]
    w2T = [p["nn_W2"][l].T for l in range(NL)]
    b2T = [p["nn_b2"][l][None, :] for l in range(NL)]
    mw1T = [p["mlp_W1"][l].T for l in range(NL)]
    mb1T = [p["mlp_b1"][l][None, :] for l in range(NL)]
    mw2T = [p["mlp_W2"][l].T for l in range(NL)]
    mb2T = [p["mlp_b2"][l][None, :] for l in range(NL)]
    ng = {nm: [p[nm + "_g"][l][None, :] for l in range(NL)] for nm in ("n1", "n2", "n3")}
    nb = {nm: [p[nm + "_b"][l][None, :] for l in range(NL)] for nm in ("n1", "n2", "n3")}

    pad_e = ERP * 128 - E

    def _erp(v, dt, fill=0):
        return jnp.pad(v.astype(dt), (0, pad_e),
                       constant_values=fill).reshape(ERP, 128)

    acols = [_erp(edge_attr[:, j], F32) for j in range(4)]
    src2 = _erp(src, jnp.int32)
    dst2 = _erp(dst, jnp.int32, fill=N)

    seg_off = jnp.searchsorted(batch32, jnp.arange(G + 1, dtype=jnp.int32)).astype(jnp.int32)
    qlo2 = seg_off[batch32][:, None]
    qhi2 = seg_off[batch32 + 1][:, None]
    g_first = batch32[::TQ]
    g_last = batch32[TQ - 1::TQ]
    kstart = (seg_off[g_first] // TK) * TK
    nkt = (seg_off[g_last + 1] - kstart + TK - 1) // TK
    bat3 = batch32.reshape(NMT, 1, BT)
    zero_wb = jnp.zeros((WB, 32), F32)

    gid2 = pl.pallas_call(
        _eidx_body,
        out_shape=jax.ShapeDtypeStruct((ERP, 128), jnp.int32),
    )(*acols, src2)

    row_spec = pl.BlockSpec((BT, C), lambda i: (i, _Z()))
    kv_spec = pl.BlockSpec((BT, 2 * C), lambda i: (i, _Z()))
    r_spec = pl.BlockSpec((4, BT, 32), lambda i: (_Z(), i, _Z()))
    nodes_f32 = jax.ShapeDtypeStruct((N, C), F32)
    kv_f32 = jax.ShapeDtypeStruct((N, 2 * C), F32)
    rtab = jax.ShapeDtypeStruct((4, N, 32), F32)

    h, q, kv, ra, rb = pl.pallas_call(
        _k0_body,
        grid=(NMT,),
        in_specs=[pl.BlockSpec((BT, 32), lambda i: (i, _Z())),
                  _const_spec((32, C)), _const_spec((1, C)),
                  _const_spec((C, 3 * C)), _const_spec((1, 3 * C)),
                  _const_spec((4, C))],
        out_specs=[row_spec, row_spec, kv_spec] + [r_spec] * 2,
        out_shape=[nodes_f32, nodes_f32, kv_f32] + [rtab] * 2,
    )(X, w0, b0, wiT[0], biT[0], emb)

    pooled = None
    for l in range(NL):
        agg2 = _sc_edge_agg(gid2, dst2, ra.reshape(4 * N, 32),
                            rb.reshape(4 * N, 32), zero_wb)
        agg3 = agg2.reshape(2, NP, 32)

        h2 = pl.pallas_call(
            _attn_body,
            grid=(NQT,),
            in_specs=[pl.BlockSpec((NQT,), lambda i: (_Z(),),
                                   memory_space=pltpu.SMEM),
                      pl.BlockSpec((NQT,), lambda i: (_Z(),),
                                   memory_space=pltpu.SMEM),
                      pl.BlockSpec((TQ, C), lambda i: (i, _Z())),
                      pl.BlockSpec((TQ, 1), lambda i: (i, _Z())),
                      pl.BlockSpec((TQ, 1), lambda i: (i, _Z())),
                      pl.BlockSpec((TQ, C), lambda i: (i, _Z())),
                      _const_spec((C, C)), _const_spec((1, C)),
                      _const_spec((1, C)), _const_spec((1, C)),
                      pl.BlockSpec(memory_space=pl.ANY)],
            out_specs=pl.BlockSpec((TQ, C), lambda i: (i, _Z())),
            out_shape=nodes_f32,
            scratch_shapes=[pltpu.VMEM((2, TK, 2 * C), F32),
                            pltpu.SemaphoreType.DMA((2,))],
        )(kstart, nkt, q, qlo2, qhi2, h, woT[l], boT[l],
          ng["n2"][l], nb["n2"][l], kv)

        mid_w = (w1T[l], b1T[l], w2T[l], b2T[l],
                 mw1T[l], mb1T[l], mw2T[l], mb2T[l],
                 ng["n1"][l], nb["n1"][l], ng["n3"][l], nb["n3"][l])
        mid_w_specs = [_const_spec((C, C)), _const_spec((1, C)),
                       _const_spec((C, C)), _const_spec((1, C)),
                       _const_spec((C, 2 * C)), _const_spec((1, 2 * C)),
                       _const_spec((2 * C, C)), _const_spec((1, C)),
                       _const_spec((1, C)), _const_spec((1, C)),
                       _const_spec((1, C)), _const_spec((1, C))]
        agg_spec = pl.BlockSpec((2, BT, 32), lambda i: (_Z(), i, _Z()))

        if l < NL - 1:
            h, q, kv, ra, rb = pl.pallas_call(
                _mid_body,
                grid=(NMT,),
                in_specs=[row_spec, agg_spec, row_spec] + mid_w_specs +
                         [_const_spec((C, 3 * C)), _const_spec((1, 3 * C)),
                          _const_spec((4, C))],
                out_specs=[row_spec, row_spec, kv_spec] + [r_spec] * 2,
                out_shape=[nodes_f32, nodes_f32, kv_f32] + [rtab] * 2,
            )(h, agg3, h2, *mid_w, wiT[l + 1], biT[l + 1], emb)
        else:
            pooled = pl.pallas_call(
                _pool_body,
                grid=(NMT,),
                in_specs=[row_spec, agg_spec, row_spec] + mid_w_specs +
                         [pl.BlockSpec((1, 1, BT), lambda i: (i, _Z(), _Z()))],
                out_specs=pl.BlockSpec((G, C), lambda i: (_Z(), _Z())),
                out_shape=jax.ShapeDtypeStruct((G, C), F32),
            )(h, agg3, h2, *mid_w, bat3)

    return pooled


# speedup vs baseline: 29.9986x; 1.2154x over previous
import functools

import jax
import jax.numpy as jnp
from jax import lax
from jax.experimental import pallas as pl
from jax.experimental.pallas import tpu as pltpu
from jax.experimental.pallas import tpu_sc as plsc

N = 50000
E = 800000
G = 1000
C = 64
H = 4
DH = 16
NL = 3

TQ = 400
TK = 400
NQT = N // TQ

BT = 1000
NMT = N // BT

ER = E // 128
ERP = 6256
EGB = 16
NGRP = ERP // EGB

NSUB = 16
NP = 50048
NZR = NP // NSUB
WB = 184
NWB = NZR // WB

NEG = -1e9
F32 = jnp.float32


def _eidx_body(a0, a1, a2, a3, s_ref, gid_ref):
    best = a0[...]
    bi = jnp.zeros(best.shape, jnp.int32)
    for j, ar in enumerate((a1, a2, a3), start=1):
        v = ar[...]
        bi = jnp.where(v > best, j, bi)
        best = jnp.maximum(best, v)
    gid_ref[...] = bi * N + s_ref[...]


def _emit_pre(h, wi_ref, bi_ref, emb_ref, q_ref, kv_ref, ra_ref, rb_ref):
    qkv = jnp.dot(h, wi_ref[...], preferred_element_type=F32) + bi_ref[...]
    q_ref[...] = qkv[:, 0:C]
    kv_ref[...] = qkv[:, C:3 * C]
    for cc in range(4):
        rc = jnp.maximum(h + emb_ref[cc:cc + 1, :], 0.0)
        ra_ref[cc, :, :] = rc[:, :32]
        rb_ref[cc, :, :] = rc[:, 32:]


def _k0_body(x_ref, w0_ref, b0_ref, wi_ref, bi_ref, emb_ref,
             h_ref, q_ref, kv_ref, ra_ref, rb_ref):
    h = jnp.dot(x_ref[...], w0_ref[...], preferred_element_type=F32) + b0_ref[...]
    h_ref[...] = h
    _emit_pre(h, wi_ref, bi_ref, emb_ref, q_ref, kv_ref, ra_ref, rb_ref)


def _mid_common(h_ref, agg_ref, h2_ref, w1, b1, w2, b2, mw1, mb1, mw2, mb2,
                n1g, n1b, n3g, n3b):
    h = h_ref[...]
    ag = agg_ref[...]
    agg = jnp.concatenate([ag[0], ag[1]], axis=1)
    t = h + agg
    t = jnp.dot(jnp.maximum(jnp.dot(t, w1[...], preferred_element_type=F32)
                            + b1[...], 0.0),
                w2[...], preferred_element_type=F32) + b2[...]
    h1 = (t + h) * n1g[...] + n1b[...]
    out = h1 + h2_ref[...]
    mlp = jnp.dot(jnp.maximum(jnp.dot(out, mw1[...], preferred_element_type=F32)
                              + mb1[...], 0.0),
                  mw2[...], preferred_element_type=F32) + mb2[...]
    return (out + mlp) * n3g[...] + n3b[...]


def _mid_body(h_ref, agg_ref, h2_ref, w1, b1, w2, b2, mw1, mb1, mw2, mb2,
              n1g, n1b, n3g, n3b, wi_ref, bi_ref, emb_ref,
              hn_ref, q_ref, kv_ref, ra_ref, rb_ref):
    hn = _mid_common(h_ref, agg_ref, h2_ref, w1, b1, w2, b2,
                     mw1, mb1, mw2, mb2, n1g, n1b, n3g, n3b)
    hn_ref[...] = hn
    _emit_pre(hn, wi_ref, bi_ref, emb_ref, q_ref, kv_ref, ra_ref, rb_ref)


def _pool_body(h_ref, agg_ref, h2_ref, w1, b1, w2, b2, mw1, mb1, mw2, mb2,
               n1g, n1b, n3g, n3b, bat_ref, pool_ref):
    hn = _mid_common(h_ref, agg_ref, h2_ref, w1, b1, w2, b2,
                     mw1, mb1, mw2, mb2, n1g, n1b, n3g, n3b)
    bt = bat_ref[0]
    oh = (lax.broadcasted_iota(jnp.int32, (G, BT), 0) == bt).astype(F32)
    contrib = jnp.dot(oh, hn, preferred_element_type=F32)

    @pl.when(pl.program_id(0) == 0)
    def _init():
        pool_ref[...] = jnp.zeros_like(pool_ref)

    pool_ref[...] = pool_ref[...] + contrib


def _attn_body(ks_ref, nk_ref, q_ref, qlo_ref, qhi_ref, h_ref,
               wo_ref, bo_ref, g_ref, bb_ref, kv_hbm,
               o_ref, kvbuf, sems):
    i = pl.program_id(0)
    start = ks_ref[i]
    nk = nk_ref[i]
    qlo = qlo_ref[0]
    qhi = qhi_ref[0]

    def desc(kt, par):
        off = start + kt * TK
        return pltpu.make_async_copy(kv_hbm.at[pl.ds(off, TK)],
                                     kvbuf.at[par], sems.at[par])

    desc(jnp.int32(0), jnp.int32(0)).start()

    def body(kt, carry):
        ms, ls, accs = carry
        par = lax.rem(kt, jnp.int32(2))

        @pl.when(kt + 1 < nk)
        def _pf():
            desc(kt + 1, 1 - par).start()

        desc(kt, par).wait()
        kv = kvbuf[par]
        off = start + kt * TK
        col = off + lax.broadcasted_iota(jnp.int32, (TK, TQ), 0)
        valid = (col >= qlo) & (col < qhi)
        nms, nls, naccs = [], [], []
        for hh in range(H):
            qs = q_ref[:, DH * hh:DH * (hh + 1)] * 0.25
            st = lax.dot_general(kv[:, DH * hh:DH * (hh + 1)], qs,
                                 (((1,), (1,)), ((), ())),
                                 preferred_element_type=F32)
            st = jnp.where(valid, st, NEG)
            mn = jnp.maximum(ms[hh], jnp.max(st, axis=0, keepdims=True))
            al = jnp.exp(ms[hh] - mn)
            pt = jnp.exp(st - mn)
            nls.append(ls[hh] * al + jnp.sum(pt, axis=0, keepdims=True))
            pv = lax.dot_general(kv[:, C + DH * hh:C + DH * (hh + 1)], pt,
                                 (((0,), (0,)), ((), ())),
                                 preferred_element_type=F32)
            naccs.append(accs[hh] * al + pv)
            nms.append(mn)
        return tuple(nms), tuple(nls), tuple(naccs)

    m0 = tuple(jnp.full((1, TQ), -1e30, F32) for _ in range(H))
    l0 = tuple(jnp.zeros((1, TQ), F32) for _ in range(H))
    a0 = tuple(jnp.zeros((DH, TQ), F32) for _ in range(H))
    ms, ls, accs = lax.fori_loop(jnp.int32(0), nk, body, (m0, l0, a0))
    ot = jnp.concatenate([accs[hh] / ls[hh] for hh in range(H)], axis=0)
    a_out = lax.dot_general(ot, wo_ref[...], (((0,), (0,)), ((), ())),
                            preferred_element_type=F32) + bo_ref[...]
    o_ref[...] = (a_out + h_ref[...]) * g_ref[...] + bb_ref[...]


_sc_mesh = plsc.VectorSubcoreMesh(core_axis_name="c", subcore_axis_name="s")


@functools.partial(
    pl.kernel,
    mesh=_sc_mesh,
    compiler_params=pltpu.CompilerParams(use_tc_tiling_on_sc=False),
    out_type=jax.ShapeDtypeStruct((2 * NP, 32), F32),
    scratch_types=[
        pltpu.VMEM((EGB, 128), jnp.int32),
        pltpu.VMEM((EGB, 128), jnp.int32),
        pltpu.VMEM((2, 2, 128, 32), F32),
        pltpu.VMEM((WB, 32), F32),
        pltpu.VMEM_SHARED((NP, 32), F32),
        pltpu.SemaphoreType.DMA,
        pltpu.SemaphoreType.DMA,
    ],
)
def _sc_edge_agg(gid_hbm, dst_hbm, r0_hbm, r1_hbm, zero_hbm, out_hbm,
                 gbuf, dbuf, rows, zbuf, aggsh, gsem, ssem):
    c = lax.axis_index("c")
    s = lax.axis_index("s")

    pltpu.sync_copy(zero_hbm, zbuf)
    base = s * NZR
    for j in range(NWB):
        pltpu.sync_copy(zbuf, aggsh.at[pl.ds(base + j * WB, WB)])
    plsc.subcore_barrier()

    ngrp = jnp.int32(NGRP // NSUB) + (s < NGRP % NSUB).astype(jnp.int32)
    nsub = EGB // 2

    def body(i, carry):
        g = s + i * NSUB
        pltpu.sync_copy(gid_hbm.at[pl.ds(g * EGB, EGB)], gbuf)
        pltpu.sync_copy(dst_hbm.at[pl.ds(g * EGB, EGB)], dbuf)

        def pipe(r_hbm):
            def gfire(t):
                p = t % 2
                return [pltpu.async_copy(
                    r_hbm.at[gbuf.at[jnp.int32(2 * t + j)]],
                    rows.at[jnp.int32(p), jnp.int32(j)], gsem)
                    for j in range(2)]

            def sfire(t):
                p = t % 2
                return [pltpu.async_copy(
                    rows.at[jnp.int32(p), jnp.int32(j)],
                    aggsh.at[dbuf.at[jnp.int32(2 * t + j)]], ssem, add=True)
                    for j in range(2)]

            gd = gfire(0)
            sd_prev = None
            for t in range(nsub):
                for d in gd:
                    d.wait()
                sd = sfire(t)
                if sd_prev is not None:
                    for d in sd_prev:
                        d.wait()
                if t < nsub - 1:
                    gd = gfire(t + 1)
                sd_prev = sd
            for d in sd_prev:
                d.wait()

        @pl.when(c == 0)
        def _p0():
            pipe(r0_hbm)

        @pl.when(c == 1)
        def _p1():
            pipe(r1_hbm)

        return carry

    lax.fori_loop(jnp.int32(0), ngrp, body, jnp.int32(0))
    plsc.subcore_barrier()

    for j in range(NWB):
        pltpu.sync_copy(aggsh.at[pl.ds(base + j * WB, WB)], zbuf)
        pltpu.sync_copy(zbuf, out_hbm.at[pl.ds(c * NP + base + j * WB, WB)])


def _Z():
    return jnp.int32(0)


def _const_spec(shape):
    return pl.BlockSpec(shape, lambda i: tuple(_Z() for _ in shape))


def kernel(x, pe, edge_attr, params, edge_index, batch):
    p = params
    batch32 = batch.astype(jnp.int32)
    src = edge_index[0].astype(jnp.int32)
    dst = edge_index[1].astype(jnp.int32)

    xf = x.astype(F32)
    X = jnp.concatenate([xf, pe.astype(F32), jnp.zeros((N, 3), F32)], axis=1)
    w0 = jnp.zeros((32, C), F32)
    w0 = w0.at[0:9, 0:C - 8].set(p["node_W"].T)
    pe_w = p["pe_W"] * p["pe_norm_g"][None, :]
    pe_b = p["pe_W"] @ p["pe_norm_b"] + p["pe_b"]
    w0 = w0.at[9:29, C - 8:C].set(pe_w.T)
    b0 = jnp.concatenate([p["node_b"], pe_b])[None, :]

    emb = p["edge_emb"].astype(F32)
    wiT = [p["attn_Wi"][l].T for l in range(NL)]
    biT = [p["attn_bi"][l][None, :] for l in range(NL)]
    woT = [p["attn_Wo"][l].T for l in range(NL)]
    boT = [p["attn_bo"][l][None, :] for l in range(NL)]
    w1T = [p["nn_W1"][l].T for l in range(NL)]
    b1T = [p["nn_b1"][l][None, :] for l in range(NL)]
    w2T = [p["nn_W2"][l].T for l in range(NL)]
    b2T = [p["nn_b2"][l][None, :] for l in range(NL)]
    mw1T = [p["mlp_W1"][l].T for l in range(NL)]
    mb1T = [p["mlp_b1"][l][None, :] for l in range(NL)]
    mw2T = [p["mlp_W2"][l].T for l in range(NL)]
    mb2T = [p["mlp_b2"][l][None, :] for l in range(NL)]
    ng = {nm: [p[nm + "_g"][l][None, :] for l in range(NL)] for nm in ("n1", "n2", "n3")}
    nb = {nm: [p[nm + "_b"][l][None, :] for l in range(NL)] for nm in ("n1", "n2", "n3")}

    pad_e = ERP * 128 - E

    def _erp(v, dt, fill=0):
        return jnp.pad(v.astype(dt), (0, pad_e),
                       constant_values=fill).reshape(ERP, 128)

    acols = [_erp(edge_attr[:, j], F32) for j in range(4)]
    src2 = _erp(src, jnp.int32)
    dst2 = _erp(dst, jnp.int32, fill=N)

    seg_off = jnp.searchsorted(batch32, jnp.arange(G + 1, dtype=jnp.int32)).astype(jnp.int32)
    qlo3 = seg_off[batch32].reshape(NQT, 1, TQ)
    qhi3 = seg_off[batch32 + 1].reshape(NQT, 1, TQ)
    g_first = batch32[::TQ]
    g_last = batch32[TQ - 1::TQ]
    kstart = (seg_off[g_first] // TK) * TK
    nkt = (seg_off[g_last + 1] - kstart + TK - 1) // TK
    bat3 = batch32.reshape(NMT, 1, BT)
    zero_wb = jnp.zeros((WB, 32), F32)

    gid2 = pl.pallas_call(
        _eidx_body,
        out_shape=jax.ShapeDtypeStruct((ERP, 128), jnp.int32),
    )(*acols, src2)

    row_spec = pl.BlockSpec((BT, C), lambda i: (i, _Z()))
    kv_spec = pl.BlockSpec((BT, 2 * C), lambda i: (i, _Z()))
    r_spec = pl.BlockSpec((4, BT, 32), lambda i: (_Z(), i, _Z()))
    nodes_f32 = jax.ShapeDtypeStruct((N, C), F32)
    kv_f32 = jax.ShapeDtypeStruct((N, 2 * C), F32)
    rtab = jax.ShapeDtypeStruct((4, N, 32), F32)

    h, q, kv, ra, rb = pl.pallas_call(
        _k0_body,
        grid=(NMT,),
        in_specs=[pl.BlockSpec((BT, 32), lambda i: (i, _Z())),
                  _const_spec((32, C)), _const_spec((1, C)),
                  _const_spec((C, 3 * C)), _const_spec((1, 3 * C)),
                  _const_spec((4, C))],
        out_specs=[row_spec, row_spec, kv_spec] + [r_spec] * 2,
        out_shape=[nodes_f32, nodes_f32, kv_f32] + [rtab] * 2,
    )(X, w0, b0, wiT[0], biT[0], emb)

    pooled = None
    for l in range(NL):
        agg2 = _sc_edge_agg(gid2, dst2, ra.reshape(4 * N, 32),
                            rb.reshape(4 * N, 32), zero_wb)
        agg3 = agg2.reshape(2, NP, 32)

        h2 = pl.pallas_call(
            _attn_body,
            grid=(NQT,),
            in_specs=[pl.BlockSpec((NQT,), lambda i: (_Z(),),
                                   memory_space=pltpu.SMEM),
                      pl.BlockSpec((NQT,), lambda i: (_Z(),),
                                   memory_space=pltpu.SMEM),
                      pl.BlockSpec((TQ, C), lambda i: (i, _Z())),
                      pl.BlockSpec((1, 1, TQ), lambda i: (i, _Z(), _Z())),
                      pl.BlockSpec((1, 1, TQ), lambda i: (i, _Z(), _Z())),
                      pl.BlockSpec((TQ, C), lambda i: (i, _Z())),
                      _const_spec((C, C)), _const_spec((1, C)),
                      _const_spec((1, C)), _const_spec((1, C)),
                      pl.BlockSpec(memory_space=pl.ANY)],
            out_specs=pl.BlockSpec((TQ, C), lambda i: (i, _Z())),
            out_shape=nodes_f32,
            scratch_shapes=[pltpu.VMEM((2, TK, 2 * C), F32),
                            pltpu.SemaphoreType.DMA((2,))],
        )(kstart, nkt, q, qlo3, qhi3, h, woT[l], boT[l],
          ng["n2"][l], nb["n2"][l], kv)

        mid_w = (w1T[l], b1T[l], w2T[l], b2T[l],
                 mw1T[l], mb1T[l], mw2T[l], mb2T[l],
                 ng["n1"][l], nb["n1"][l], ng["n3"][l], nb["n3"][l])
        mid_w_specs = [_const_spec((C, C)), _const_spec((1, C)),
                       _const_spec((C, C)), _const_spec((1, C)),
                       _const_spec((C, 2 * C)), _const_spec((1, 2 * C)),
                       _const_spec((2 * C, C)), _const_spec((1, C)),
                       _const_spec((1, C)), _const_spec((1, C)),
                       _const_spec((1, C)), _const_spec((1, C))]
        agg_spec = pl.BlockSpec((2, BT, 32), lambda i: (_Z(), i, _Z()))

        if l < NL - 1:
            h, q, kv, ra, rb = pl.pallas_call(
                _mid_body,
                grid=(NMT,),
                in_specs=[row_spec, agg_spec, row_spec] + mid_w_specs +
                         [_const_spec((C, 3 * C)), _const_spec((1, 3 * C)),
                          _const_spec((4, C))],
                out_specs=[row_spec, row_spec, kv_spec] + [r_spec] * 2,
                out_shape=[nodes_f32, nodes_f32, kv_f32] + [rtab] * 2,
            )(h, agg3, h2, *mid_w, wiT[l + 1], biT[l + 1], emb)
        else:
            pooled = pl.pallas_call(
                _pool_body,
                grid=(NMT,),
                in_specs=[row_spec, agg_spec, row_spec] + mid_w_specs +
                         [pl.BlockSpec((1, 1, BT), lambda i: (i, _Z(), _Z()))],
                out_specs=pl.BlockSpec((G, C), lambda i: (_Z(), _Z())),
                out_shape=jax.ShapeDtypeStruct((G, C), F32),
            )(h, agg3, h2, *mid_w, bat3)

    return pooled


# speedup vs baseline: 30.8111x; 1.0271x over previous
import functools

import jax
import jax.numpy as jnp
from jax import lax
from jax.experimental import pallas as pl
from jax.experimental.pallas import tpu as pltpu
from jax.experimental.pallas import tpu_sc as plsc

N = 50000
E = 800000
G = 1000
C = 64
H = 4
DH = 16
NL = 3

TQ = 400
TK = 400
NQT = N // TQ

BT = 1000
NMT = N // BT

ER = E // 128
ERP = 6256
EGB = 16
NGRP = ERP // EGB

NSUB = 16
NP = 50048
NZR = NP // NSUB
WB = 184
NWB = NZR // WB

NEG = -1e9
F32 = jnp.float32


def _eidx_body(a0, a1, a2, a3, s_ref, gid_ref):
    best = a0[...]
    bi = jnp.zeros(best.shape, jnp.int32)
    for j, ar in enumerate((a1, a2, a3), start=1):
        v = ar[...]
        bi = jnp.where(v > best, j, bi)
        best = jnp.maximum(best, v)
    gid_ref[...] = bi * N + s_ref[...]


def _emit_pre(h, wi_ref, bi_ref, emb_ref, q_ref, kv_ref, ra_ref, rb_ref):
    qkv = jnp.dot(h, wi_ref[...], preferred_element_type=F32) + bi_ref[...]
    q_ref[...] = qkv[:, 0:C]
    kv_ref[...] = qkv[:, C:3 * C]
    for cc in range(4):
        rc = jnp.maximum(h + emb_ref[cc:cc + 1, :], 0.0)
        ra_ref[cc, :, :] = rc[:, :32]
        rb_ref[cc, :, :] = rc[:, 32:]


def _k0_body(x_ref, w0_ref, b0_ref, wi_ref, bi_ref, emb_ref,
             h_ref, q_ref, kv_ref, ra_ref, rb_ref):
    h = jnp.dot(x_ref[...], w0_ref[...], preferred_element_type=F32) + b0_ref[...]
    h_ref[...] = h
    _emit_pre(h, wi_ref, bi_ref, emb_ref, q_ref, kv_ref, ra_ref, rb_ref)


def _mid_common(h_ref, agg_ref, h2_ref, w1, b1, w2, b2, mw1, mb1, mw2, mb2,
                n1g, n1b, n3g, n3b):
    h = h_ref[...]
    ag = agg_ref[...]
    agg = jnp.concatenate([ag[0], ag[1]], axis=1)
    t = h + agg
    t = jnp.dot(jnp.maximum(jnp.dot(t, w1[...], preferred_element_type=F32)
                            + b1[...], 0.0),
                w2[...], preferred_element_type=F32) + b2[...]
    h1 = (t + h) * n1g[...] + n1b[...]
    out = h1 + h2_ref[...]
    mlp = jnp.dot(jnp.maximum(jnp.dot(out, mw1[...], preferred_element_type=F32)
                              + mb1[...], 0.0),
                  mw2[...], preferred_element_type=F32) + mb2[...]
    return (out + mlp) * n3g[...] + n3b[...]


def _mid_body(h_ref, agg_ref, h2_ref, w1, b1, w2, b2, mw1, mb1, mw2, mb2,
              n1g, n1b, n3g, n3b, wi_ref, bi_ref, emb_ref,
              hn_ref, q_ref, kv_ref, ra_ref, rb_ref):
    hn = _mid_common(h_ref, agg_ref, h2_ref, w1, b1, w2, b2,
                     mw1, mb1, mw2, mb2, n1g, n1b, n3g, n3b)
    hn_ref[...] = hn
    _emit_pre(hn, wi_ref, bi_ref, emb_ref, q_ref, kv_ref, ra_ref, rb_ref)


def _pool_body(h_ref, agg_ref, h2_ref, w1, b1, w2, b2, mw1, mb1, mw2, mb2,
               n1g, n1b, n3g, n3b, bat_ref, pool_ref):
    hn = _mid_common(h_ref, agg_ref, h2_ref, w1, b1, w2, b2,
                     mw1, mb1, mw2, mb2, n1g, n1b, n3g, n3b)
    bt = bat_ref[0]
    oh = (lax.broadcasted_iota(jnp.int32, (G, BT), 0) == bt).astype(F32)
    contrib = jnp.dot(oh, hn, preferred_element_type=F32)

    @pl.when(pl.program_id(0) == 0)
    def _init():
        pool_ref[...] = jnp.zeros_like(pool_ref)

    pool_ref[...] = pool_ref[...] + contrib


def _attn_body(ks_ref, nk_ref, q_ref, qlo_ref, qhi_ref, h_ref,
               wo_ref, bo_ref, g_ref, bb_ref, kv_hbm,
               o_ref, kvbuf, sems):
    i = pl.program_id(0)
    start = ks_ref[i]
    nk = nk_ref[i]
    qlo = qlo_ref[0]
    qhi = qhi_ref[0]

    def desc(kt, par):
        off = start + kt * TK
        return pltpu.make_async_copy(kv_hbm.at[pl.ds(off, TK)],
                                     kvbuf.at[par], sems.at[par])

    desc(jnp.int32(0), jnp.int32(0)).start()

    def body(kt, carry):
        ms, ls, accs = carry
        par = lax.rem(kt, jnp.int32(2))

        @pl.when(kt + 1 < nk)
        def _pf():
            desc(kt + 1, 1 - par).start()

        desc(kt, par).wait()
        kv = kvbuf[par]
        off = start + kt * TK
        col = off + lax.broadcasted_iota(jnp.int32, (TK, TQ), 0)
        madd = jnp.where((col >= qlo) & (col < qhi),
                         jnp.float32(0.0), jnp.float32(NEG))
        nms, nls, naccs = [], [], []
        for hh in range(H):
            qs = q_ref[:, DH * hh:DH * (hh + 1)] * 0.25
            st = lax.dot_general(kv[:, DH * hh:DH * (hh + 1)], qs,
                                 (((1,), (1,)), ((), ())),
                                 preferred_element_type=F32)
            st = st + madd
            mn = jnp.maximum(ms[hh], jnp.max(st, axis=0, keepdims=True))
            al = jnp.exp(ms[hh] - mn)
            pt = jnp.exp(st - mn)
            nls.append(ls[hh] * al + jnp.sum(pt, axis=0, keepdims=True))
            pv = lax.dot_general(kv[:, C + DH * hh:C + DH * (hh + 1)], pt,
                                 (((0,), (0,)), ((), ())),
                                 preferred_element_type=F32)
            naccs.append(accs[hh] * al + pv)
            nms.append(mn)
        return tuple(nms), tuple(nls), tuple(naccs)

    m0 = tuple(jnp.full((1, TQ), -1e30, F32) for _ in range(H))
    l0 = tuple(jnp.zeros((1, TQ), F32) for _ in range(H))
    a0 = tuple(jnp.zeros((DH, TQ), F32) for _ in range(H))
    ms, ls, accs = lax.fori_loop(jnp.int32(0), nk, body, (m0, l0, a0))
    ot = jnp.concatenate([accs[hh] / ls[hh] for hh in range(H)], axis=0)
    a_out = lax.dot_general(ot, wo_ref[...], (((0,), (0,)), ((), ())),
                            preferred_element_type=F32) + bo_ref[...]
    o_ref[...] = (a_out + h_ref[...]) * g_ref[...] + bb_ref[...]


_sc_mesh = plsc.VectorSubcoreMesh(core_axis_name="c", subcore_axis_name="s")


@functools.partial(
    pl.kernel,
    mesh=_sc_mesh,
    compiler_params=pltpu.CompilerParams(use_tc_tiling_on_sc=False),
    out_type=jax.ShapeDtypeStruct((2 * NP, 32), F32),
    scratch_types=[
        pltpu.VMEM((EGB, 128), jnp.int32),
        pltpu.VMEM((EGB, 128), jnp.int32),
        pltpu.VMEM((2, 2, 128, 32), F32),
        pltpu.VMEM((WB, 32), F32),
        pltpu.VMEM_SHARED((NP, 32), F32),
        pltpu.SemaphoreType.DMA,
        pltpu.SemaphoreType.DMA,
    ],
)
def _sc_edge_agg(gid_hbm, dst_hbm, r0_hbm, r1_hbm, zero_hbm, out_hbm,
                 gbuf, dbuf, rows, zbuf, aggsh, gsem, ssem):
    c = lax.axis_index("c")
    s = lax.axis_index("s")

    pltpu.sync_copy(zero_hbm, zbuf)
    base = s * NZR
    for j in range(NWB):
        pltpu.sync_copy(zbuf, aggsh.at[pl.ds(base + j * WB, WB)])
    plsc.subcore_barrier()

    ngrp = jnp.int32(NGRP // NSUB) + (s < NGRP % NSUB).astype(jnp.int32)
    nsub = EGB // 2

    def body(i, carry):
        g = s + i * NSUB
        pltpu.sync_copy(gid_hbm.at[pl.ds(g * EGB, EGB)], gbuf)
        pltpu.sync_copy(dst_hbm.at[pl.ds(g * EGB, EGB)], dbuf)

        def pipe(r_hbm):
            def gfire(t):
                p = t % 2
                return [pltpu.async_copy(
                    r_hbm.at[gbuf.at[jnp.int32(2 * t + j)]],
                    rows.at[jnp.int32(p), jnp.int32(j)], gsem)
                    for j in range(2)]

            def sfire(t):
                p = t % 2
                return [pltpu.async_copy(
                    rows.at[jnp.int32(p), jnp.int32(j)],
                    aggsh.at[dbuf.at[jnp.int32(2 * t + j)]], ssem, add=True)
                    for j in range(2)]

            gd = gfire(0)
            sd_prev = None
            for t in range(nsub):
                for d in gd:
                    d.wait()
                sd = sfire(t)
                if sd_prev is not None:
                    for d in sd_prev:
                        d.wait()
                if t < nsub - 1:
                    gd = gfire(t + 1)
                sd_prev = sd
            for d in sd_prev:
                d.wait()

        @pl.when(c == 0)
        def _p0():
            pipe(r0_hbm)

        @pl.when(c == 1)
        def _p1():
            pipe(r1_hbm)

        return carry

    lax.fori_loop(jnp.int32(0), ngrp, body, jnp.int32(0))
    plsc.subcore_barrier()

    for j in range(NWB):
        pltpu.sync_copy(aggsh.at[pl.ds(base + j * WB, WB)], zbuf)
        pltpu.sync_copy(zbuf, out_hbm.at[pl.ds(c * NP + base + j * WB, WB)])


def _Z():
    return jnp.int32(0)


def _const_spec(shape):
    return pl.BlockSpec(shape, lambda i: tuple(_Z() for _ in shape))


def kernel(x, pe, edge_attr, params, edge_index, batch):
    p = params
    batch32 = batch.astype(jnp.int32)
    src = edge_index[0].astype(jnp.int32)
    dst = edge_index[1].astype(jnp.int32)

    xf = x.astype(F32)
    X = jnp.concatenate([xf, pe.astype(F32), jnp.zeros((N, 3), F32)], axis=1)
    w0 = jnp.zeros((32, C), F32)
    w0 = w0.at[0:9, 0:C - 8].set(p["node_W"].T)
    pe_w = p["pe_W"] * p["pe_norm_g"][None, :]
    pe_b = p["pe_W"] @ p["pe_norm_b"] + p["pe_b"]
    w0 = w0.at[9:29, C - 8:C].set(pe_w.T)
    b0 = jnp.concatenate([p["node_b"], pe_b])[None, :]

    emb = p["edge_emb"].astype(F32)
    wiT = [p["attn_Wi"][l].T for l in range(NL)]
    biT = [p["attn_bi"][l][None, :] for l in range(NL)]
    woT = [p["attn_Wo"][l].T for l in range(NL)]
    boT = [p["attn_bo"][l][None, :] for l in range(NL)]
    w1T = [p["nn_W1"][l].T for l in range(NL)]
    b1T = [p["nn_b1"][l][None, :] for l in range(NL)]
    w2T = [p["nn_W2"][l].T for l in range(NL)]
    b2T = [p["nn_b2"][l][None, :] for l in range(NL)]
    mw1T = [p["mlp_W1"][l].T for l in range(NL)]
    mb1T = [p["mlp_b1"][l][None, :] for l in range(NL)]
    mw2T = [p["mlp_W2"][l].T for l in range(NL)]
    mb2T = [p["mlp_b2"][l][None, :] for l in range(NL)]
    ng = {nm: [p[nm + "_g"][l][None, :] for l in range(NL)] for nm in ("n1", "n2", "n3")}
    nb = {nm: [p[nm + "_b"][l][None, :] for l in range(NL)] for nm in ("n1", "n2", "n3")}

    pad_e = ERP * 128 - E

    def _erp(v, dt, fill=0):
        return jnp.pad(v.astype(dt), (0, pad_e),
                       constant_values=fill).reshape(ERP, 128)

    acols = [_erp(edge_attr[:, j], F32) for j in range(4)]
    src2 = _erp(src, jnp.int32)
    dst2 = _erp(dst, jnp.int32, fill=N)

    seg_off = jnp.searchsorted(batch32, jnp.arange(G + 1, dtype=jnp.int32)).astype(jnp.int32)
    qlo3 = seg_off[batch32].reshape(NQT, 1, TQ)
    qhi3 = seg_off[batch32 + 1].reshape(NQT, 1, TQ)
    g_first = batch32[::TQ]
    g_last = batch32[TQ - 1::TQ]
    kstart = (seg_off[g_first] // TK) * TK
    nkt = (seg_off[g_last + 1] - kstart + TK - 1) // TK
    bat3 = batch32.reshape(NMT, 1, BT)
    zero_wb = jnp.zeros((WB, 32), F32)

    gid2 = pl.pallas_call(
        _eidx_body,
        out_shape=jax.ShapeDtypeStruct((ERP, 128), jnp.int32),
    )(*acols, src2)

    row_spec = pl.BlockSpec((BT, C), lambda i: (i, _Z()))
    kv_spec = pl.BlockSpec((BT, 2 * C), lambda i: (i, _Z()))
    r_spec = pl.BlockSpec((4, BT, 32), lambda i: (_Z(), i, _Z()))
    nodes_f32 = jax.ShapeDtypeStruct((N, C), F32)
    kv_f32 = jax.ShapeDtypeStruct((N, 2 * C), F32)
    rtab = jax.ShapeDtypeStruct((4, N, 32), F32)

    h, q, kv, ra, rb = pl.pallas_call(
        _k0_body,
        grid=(NMT,),
        in_specs=[pl.BlockSpec((BT, 32), lambda i: (i, _Z())),
                  _const_spec((32, C)), _const_spec((1, C)),
                  _const_spec((C, 3 * C)), _const_spec((1, 3 * C)),
                  _const_spec((4, C))],
        out_specs=[row_spec, row_spec, kv_spec] + [r_spec] * 2,
        out_shape=[nodes_f32, nodes_f32, kv_f32] + [rtab] * 2,
    )(X, w0, b0, wiT[0], biT[0], emb)

    pooled = None
    for l in range(NL):
        agg2 = _sc_edge_agg(gid2, dst2, ra.reshape(4 * N, 32),
                            rb.reshape(4 * N, 32), zero_wb)
        agg3 = agg2.reshape(2, NP, 32)

        h2 = pl.pallas_call(
            _attn_body,
            grid=(NQT,),
            in_specs=[pl.BlockSpec((NQT,), lambda i: (_Z(),),
                                   memory_space=pltpu.SMEM),
                      pl.BlockSpec((NQT,), lambda i: (_Z(),),
                                   memory_space=pltpu.SMEM),
                      pl.BlockSpec((TQ, C), lambda i: (i, _Z())),
                      pl.BlockSpec((1, 1, TQ), lambda i: (i, _Z(), _Z())),
                      pl.BlockSpec((1, 1, TQ), lambda i: (i, _Z(), _Z())),
                      pl.BlockSpec((TQ, C), lambda i: (i, _Z())),
                      _const_spec((C, C)), _const_spec((1, C)),
                      _const_spec((1, C)), _const_spec((1, C)),
                      pl.BlockSpec(memory_space=pl.ANY)],
            out_specs=pl.BlockSpec((TQ, C), lambda i: (i, _Z())),
            out_shape=nodes_f32,
            scratch_shapes=[pltpu.VMEM((2, TK, 2 * C), F32),
                            pltpu.SemaphoreType.DMA((2,))],
        )(kstart, nkt, q, qlo3, qhi3, h, woT[l], boT[l],
          ng["n2"][l], nb["n2"][l], kv)

        mid_w = (w1T[l], b1T[l], w2T[l], b2T[l],
                 mw1T[l], mb1T[l], mw2T[l], mb2T[l],
                 ng["n1"][l], nb["n1"][l], ng["n3"][l], nb["n3"][l])
        mid_w_specs = [_const_spec((C, C)), _const_spec((1, C)),
                       _const_spec((C, C)), _const_spec((1, C)),
                       _const_spec((C, 2 * C)), _const_spec((1, 2 * C)),
                       _const_spec((2 * C, C)), _const_spec((1, C)),
                       _const_spec((1, C)), _const_spec((1, C)),
                       _const_spec((1, C)), _const_spec((1, C))]
        agg_spec = pl.BlockSpec((2, BT, 32), lambda i: (_Z(), i, _Z()))

        if l < NL - 1:
            h, q, kv, ra, rb = pl.pallas_call(
                _mid_body,
                grid=(NMT,),
                in_specs=[row_spec, agg_spec, row_spec] + mid_w_specs +
                         [_const_spec((C, 3 * C)), _const_spec((1, 3 * C)),
                          _const_spec((4, C))],
                out_specs=[row_spec, row_spec, kv_spec] + [r_spec] * 2,
                out_shape=[nodes_f32, nodes_f32, kv_f32] + [rtab] * 2,
            )(h, agg3, h2, *mid_w, wiT[l + 1], biT[l + 1], emb)
        else:
            pooled = pl.pallas_call(
                _pool_body,
                grid=(NMT,),
                in_specs=[row_spec, agg_spec, row_spec] + mid_w_specs +
                         [pl.BlockSpec((1, 1, BT), lambda i: (i, _Z(), _Z()))],
                out_specs=pl.BlockSpec((G, C), lambda i: (_Z(), _Z())),
                out_shape=jax.ShapeDtypeStruct((G, C), F32),
            )(h, agg3, h2, *mid_w, bat3)

    return pooled


# speedup vs baseline: 36.1577x; 1.1735x over previous
import functools

import jax
import jax.numpy as jnp
from jax import lax
from jax.experimental import pallas as pl
from jax.experimental.pallas import tpu as pltpu
from jax.experimental.pallas import tpu_sc as plsc

N = 50000
E = 800000
G = 1000
C = 64
H = 4
DH = 16
NL = 3

TQ = 1000
TK = 400
NQT = N // TQ

BT = 1000
NMT = N // BT

ER = E // 128
ERP = 6256
EGB = 16
NGRP = ERP // EGB

NSUB = 16
NP = 50048
NZR = NP // NSUB
WB = 184
NWB = NZR // WB

NEG = -1e9
F32 = jnp.float32


def _eidx_body(a0, a1, a2, a3, s_ref, gid_ref):
    best = a0[...]
    bi = jnp.zeros(best.shape, jnp.int32)
    for j, ar in enumerate((a1, a2, a3), start=1):
        v = ar[...]
        bi = jnp.where(v > best, j, bi)
        best = jnp.maximum(best, v)
    gid_ref[...] = bi * N + s_ref[...]


def _emit_pre(h, wi_ref, bi_ref, emb_ref, q_ref, kv_ref, ra_ref, rb_ref):
    qkv = jnp.dot(h, wi_ref[...], preferred_element_type=F32) + bi_ref[...]
    q_ref[...] = qkv[:, 0:C]
    kv_ref[...] = qkv[:, C:3 * C]
    for cc in range(4):
        rc = jnp.maximum(h + emb_ref[cc:cc + 1, :], 0.0)
        ra_ref[cc, :, :] = rc[:, :32]
        rb_ref[cc, :, :] = rc[:, 32:]


def _k0_body(x_ref, w0_ref, b0_ref, wi_ref, bi_ref, emb_ref,
             h_ref, q_ref, kv_ref, ra_ref, rb_ref):
    h = jnp.dot(x_ref[...], w0_ref[...], preferred_element_type=F32) + b0_ref[...]
    h_ref[...] = h
    _emit_pre(h, wi_ref, bi_ref, emb_ref, q_ref, kv_ref, ra_ref, rb_ref)


def _mid_common(h_ref, agg_ref, h2_ref, w1, b1, w2, b2, mw1, mb1, mw2, mb2,
                n1g, n1b, n3g, n3b):
    h = h_ref[...]
    ag = agg_ref[...]
    agg = jnp.concatenate([ag[0], ag[1]], axis=1)
    t = h + agg
    t = jnp.dot(jnp.maximum(jnp.dot(t, w1[...], preferred_element_type=F32)
                            + b1[...], 0.0),
                w2[...], preferred_element_type=F32) + b2[...]
    h1 = (t + h) * n1g[...] + n1b[...]
    out = h1 + h2_ref[...]
    mlp = jnp.dot(jnp.maximum(jnp.dot(out, mw1[...], preferred_element_type=F32)
                              + mb1[...], 0.0),
                  mw2[...], preferred_element_type=F32) + mb2[...]
    return (out + mlp) * n3g[...] + n3b[...]


def _mid_body(h_ref, agg_ref, h2_ref, w1, b1, w2, b2, mw1, mb1, mw2, mb2,
              n1g, n1b, n3g, n3b, wi_ref, bi_ref, emb_ref,
              hn_ref, q_ref, kv_ref, ra_ref, rb_ref):
    hn = _mid_common(h_ref, agg_ref, h2_ref, w1, b1, w2, b2,
                     mw1, mb1, mw2, mb2, n1g, n1b, n3g, n3b)
    hn_ref[...] = hn
    _emit_pre(hn, wi_ref, bi_ref, emb_ref, q_ref, kv_ref, ra_ref, rb_ref)


def _pool_body(h_ref, agg_ref, h2_ref, w1, b1, w2, b2, mw1, mb1, mw2, mb2,
               n1g, n1b, n3g, n3b, bat_ref, pool_ref):
    hn = _mid_common(h_ref, agg_ref, h2_ref, w1, b1, w2, b2,
                     mw1, mb1, mw2, mb2, n1g, n1b, n3g, n3b)
    bt = bat_ref[0]
    oh = (lax.broadcasted_iota(jnp.int32, (G, BT), 0) == bt).astype(F32)
    contrib = jnp.dot(oh, hn, preferred_element_type=F32)

    @pl.when(pl.program_id(0) == 0)
    def _init():
        pool_ref[...] = jnp.zeros_like(pool_ref)

    pool_ref[...] = pool_ref[...] + contrib


def _attn_body(ks_ref, nk_ref, q_ref, qlo_ref, qhi_ref, h_ref,
               wo_ref, bo_ref, g_ref, bb_ref, kv_hbm,
               o_ref, kvbuf, sems):
    i = pl.program_id(0)
    start = ks_ref[i]
    nk = nk_ref[i]
    qlo = qlo_ref[0]
    qhi = qhi_ref[0]

    def desc(kt, par):
        off = start + kt * TK
        return pltpu.make_async_copy(kv_hbm.at[pl.ds(off, TK)],
                                     kvbuf.at[par], sems.at[par])

    desc(jnp.int32(0), jnp.int32(0)).start()

    def body(kt, carry):
        ms, ls, accs = carry
        par = lax.rem(kt, jnp.int32(2))

        @pl.when(kt + 1 < nk)
        def _pf():
            desc(kt + 1, 1 - par).start()

        desc(kt, par).wait()
        kv = kvbuf[par]
        off = start + kt * TK
        col = off + lax.broadcasted_iota(jnp.int32, (TK, TQ), 0)
        madd = jnp.where((col >= qlo) & (col < qhi),
                         jnp.float32(0.0), jnp.float32(NEG))
        nms, nls, naccs = [], [], []
        for hh in range(H):
            qs = q_ref[:, DH * hh:DH * (hh + 1)] * 0.25
            st = lax.dot_general(kv[:, DH * hh:DH * (hh + 1)], qs,
                                 (((1,), (1,)), ((), ())),
                                 preferred_element_type=F32)
            st = st + madd
            mn = jnp.maximum(ms[hh], jnp.max(st, axis=0, keepdims=True))
            al = jnp.exp(ms[hh] - mn)
            pt = jnp.exp(st - mn)
            nls.append(ls[hh] * al + jnp.sum(pt, axis=0, keepdims=True))
            pv = lax.dot_general(kv[:, C + DH * hh:C + DH * (hh + 1)], pt,
                                 (((0,), (0,)), ((), ())),
                                 preferred_element_type=F32)
            naccs.append(accs[hh] * al + pv)
            nms.append(mn)
        return tuple(nms), tuple(nls), tuple(naccs)

    m0 = tuple(jnp.full((1, TQ), -1e30, F32) for _ in range(H))
    l0 = tuple(jnp.zeros((1, TQ), F32) for _ in range(H))
    a0 = tuple(jnp.zeros((DH, TQ), F32) for _ in range(H))
    ms, ls, accs = lax.fori_loop(jnp.int32(0), nk, body, (m0, l0, a0))
    ot = jnp.concatenate([accs[hh] / ls[hh] for hh in range(H)], axis=0)
    a_out = lax.dot_general(ot, wo_ref[...], (((0,), (0,)), ((), ())),
                            preferred_element_type=F32) + bo_ref[...]
    o_ref[...] = (a_out + h_ref[...]) * g_ref[...] + bb_ref[...]


_sc_mesh = plsc.VectorSubcoreMesh(core_axis_name="c", subcore_axis_name="s")


@functools.partial(
    pl.kernel,
    mesh=_sc_mesh,
    compiler_params=pltpu.CompilerParams(use_tc_tiling_on_sc=False),
    out_type=jax.ShapeDtypeStruct((2 * NP, 32), F32),
    scratch_types=[
        pltpu.VMEM((EGB, 128), jnp.int32),
        pltpu.VMEM((EGB, 128), jnp.int32),
        pltpu.VMEM((2, 2, 128, 32), F32),
        pltpu.VMEM((WB, 32), F32),
        pltpu.VMEM_SHARED((NP, 32), F32),
        pltpu.SemaphoreType.DMA,
        pltpu.SemaphoreType.DMA,
    ],
)
def _sc_edge_agg(gid_hbm, dst_hbm, r0_hbm, r1_hbm, zero_hbm, out_hbm,
                 gbuf, dbuf, rows, zbuf, aggsh, gsem, ssem):
    c = lax.axis_index("c")
    s = lax.axis_index("s")

    pltpu.sync_copy(zero_hbm, zbuf)
    base = s * NZR
    for j in range(NWB):
        pltpu.sync_copy(zbuf, aggsh.at[pl.ds(base + j * WB, WB)])
    plsc.subcore_barrier()

    ngrp = jnp.int32(NGRP // NSUB) + (s < NGRP % NSUB).astype(jnp.int32)
    nsub = EGB // 2

    def body(i, carry):
        g = s + i * NSUB
        pltpu.sync_copy(gid_hbm.at[pl.ds(g * EGB, EGB)], gbuf)
        pltpu.sync_copy(dst_hbm.at[pl.ds(g * EGB, EGB)], dbuf)

        def pipe(r_hbm):
            def gfire(t):
                p = t % 2
                return [pltpu.async_copy(
                    r_hbm.at[gbuf.at[jnp.int32(2 * t + j)]],
                    rows.at[jnp.int32(p), jnp.int32(j)], gsem)
                    for j in range(2)]

            def sfire(t):
                p = t % 2
                return [pltpu.async_copy(
                    rows.at[jnp.int32(p), jnp.int32(j)],
                    aggsh.at[dbuf.at[jnp.int32(2 * t + j)]], ssem, add=True)
                    for j in range(2)]

            gd = gfire(0)
            sd_prev = None
            for t in range(nsub):
                for d in gd:
                    d.wait()
                sd = sfire(t)
                if sd_prev is not None:
                    for d in sd_prev:
                        d.wait()
                if t < nsub - 1:
                    gd = gfire(t + 1)
                sd_prev = sd
            for d in sd_prev:
                d.wait()

        @pl.when(c == 0)
        def _p0():
            pipe(r0_hbm)

        @pl.when(c == 1)
        def _p1():
            pipe(r1_hbm)

        return carry

    lax.fori_loop(jnp.int32(0), ngrp, body, jnp.int32(0))
    plsc.subcore_barrier()

    for j in range(NWB):
        pltpu.sync_copy(aggsh.at[pl.ds(base + j * WB, WB)], zbuf)
        pltpu.sync_copy(zbuf, out_hbm.at[pl.ds(c * NP + base + j * WB, WB)])


def _Z():
    return jnp.int32(0)


def _const_spec(shape):
    return pl.BlockSpec(shape, lambda i: tuple(_Z() for _ in shape))


def kernel(x, pe, edge_attr, params, edge_index, batch):
    p = params
    batch32 = batch.astype(jnp.int32)
    src = edge_index[0].astype(jnp.int32)
    dst = edge_index[1].astype(jnp.int32)

    xf = x.astype(F32)
    X = jnp.concatenate([xf, pe.astype(F32), jnp.zeros((N, 3), F32)], axis=1)
    w0 = jnp.zeros((32, C), F32)
    w0 = w0.at[0:9, 0:C - 8].set(p["node_W"].T)
    pe_w = p["pe_W"] * p["pe_norm_g"][None, :]
    pe_b = p["pe_W"] @ p["pe_norm_b"] + p["pe_b"]
    w0 = w0.at[9:29, C - 8:C].set(pe_w.T)
    b0 = jnp.concatenate([p["node_b"], pe_b])[None, :]

    emb = p["edge_emb"].astype(F32)
    wiT = [p["attn_Wi"][l].T for l in range(NL)]
    biT = [p["attn_bi"][l][None, :] for l in range(NL)]
    woT = [p["attn_Wo"][l].T for l in range(NL)]
    boT = [p["attn_bo"][l][None, :] for l in range(NL)]
    w1T = [p["nn_W1"][l].T for l in range(NL)]
    b1T = [p["nn_b1"][l][None, :] for l in range(NL)]
    w2T = [p["nn_W2"][l].T for l in range(NL)]
    b2T = [p["nn_b2"][l][None, :] for l in range(NL)]
    mw1T = [p["mlp_W1"][l].T for l in range(NL)]
    mb1T = [p["mlp_b1"][l][None, :] for l in range(NL)]
    mw2T = [p["mlp_W2"][l].T for l in range(NL)]
    mb2T = [p["mlp_b2"][l][None, :] for l in range(NL)]
    ng = {nm: [p[nm + "_g"][l][None, :] for l in range(NL)] for nm in ("n1", "n2", "n3")}
    nb = {nm: [p[nm + "_b"][l][None, :] for l in range(NL)] for nm in ("n1", "n2", "n3")}

    pad_e = ERP * 128 - E

    def _erp(v, dt, fill=0):
        return jnp.pad(v.astype(dt), (0, pad_e),
                       constant_values=fill).reshape(ERP, 128)

    acols = [_erp(edge_attr[:, j], F32) for j in range(4)]
    src2 = _erp(src, jnp.int32)
    dst2 = _erp(dst, jnp.int32, fill=N)

    seg_off = jnp.searchsorted(batch32, jnp.arange(G + 1, dtype=jnp.int32)).astype(jnp.int32)
    qlo3 = seg_off[batch32].reshape(NQT, 1, TQ)
    qhi3 = seg_off[batch32 + 1].reshape(NQT, 1, TQ)
    g_first = batch32[::TQ]
    g_last = batch32[TQ - 1::TQ]
    kstart = (seg_off[g_first] // TK) * TK
    nkt = (seg_off[g_last + 1] - kstart + TK - 1) // TK
    bat3 = batch32.reshape(NMT, 1, BT)
    zero_wb = jnp.zeros((WB, 32), F32)

    gid2 = pl.pallas_call(
        _eidx_body,
        out_shape=jax.ShapeDtypeStruct((ERP, 128), jnp.int32),
    )(*acols, src2)

    row_spec = pl.BlockSpec((BT, C), lambda i: (i, _Z()))
    kv_spec = pl.BlockSpec((BT, 2 * C), lambda i: (i, _Z()))
    r_spec = pl.BlockSpec((4, BT, 32), lambda i: (_Z(), i, _Z()))
    nodes_f32 = jax.ShapeDtypeStruct((N, C), F32)
    kv_f32 = jax.ShapeDtypeStruct((N, 2 * C), F32)
    rtab = jax.ShapeDtypeStruct((4, N, 32), F32)

    h, q, kv, ra, rb = pl.pallas_call(
        _k0_body,
        grid=(NMT,),
        in_specs=[pl.BlockSpec((BT, 32), lambda i: (i, _Z())),
                  _const_spec((32, C)), _const_spec((1, C)),
                  _const_spec((C, 3 * C)), _const_spec((1, 3 * C)),
                  _const_spec((4, C))],
        out_specs=[row_spec, row_spec, kv_spec] + [r_spec] * 2,
        out_shape=[nodes_f32, nodes_f32, kv_f32] + [rtab] * 2,
    )(X, w0, b0, wiT[0], biT[0], emb)

    pooled = None
    for l in range(NL):
        agg2 = _sc_edge_agg(gid2, dst2, ra.reshape(4 * N, 32),
                            rb.reshape(4 * N, 32), zero_wb)
        agg3 = agg2.reshape(2, NP, 32)

        h2 = pl.pallas_call(
            _attn_body,
            grid=(NQT,),
            in_specs=[pl.BlockSpec((NQT,), lambda i: (_Z(),),
                                   memory_space=pltpu.SMEM),
                      pl.BlockSpec((NQT,), lambda i: (_Z(),),
                                   memory_space=pltpu.SMEM),
                      pl.BlockSpec((TQ, C), lambda i: (i, _Z())),
                      pl.BlockSpec((1, 1, TQ), lambda i: (i, _Z(), _Z())),
                      pl.BlockSpec((1, 1, TQ), lambda i: (i, _Z(), _Z())),
                      pl.BlockSpec((TQ, C), lambda i: (i, _Z())),
                      _const_spec((C, C)), _const_spec((1, C)),
                      _const_spec((1, C)), _const_spec((1, C)),
                      pl.BlockSpec(memory_space=pl.ANY)],
            out_specs=pl.BlockSpec((TQ, C), lambda i: (i, _Z())),
            out_shape=nodes_f32,
            scratch_shapes=[pltpu.VMEM((2, TK, 2 * C), F32),
                            pltpu.SemaphoreType.DMA((2,))],
        )(kstart, nkt, q, qlo3, qhi3, h, woT[l], boT[l],
          ng["n2"][l], nb["n2"][l], kv)

        mid_w = (w1T[l], b1T[l], w2T[l], b2T[l],
                 mw1T[l], mb1T[l], mw2T[l], mb2T[l],
                 ng["n1"][l], nb["n1"][l], ng["n3"][l], nb["n3"][l])
        mid_w_specs = [_const_spec((C, C)), _const_spec((1, C)),
                       _const_spec((C, C)), _const_spec((1, C)),
                       _const_spec((C, 2 * C)), _const_spec((1, 2 * C)),
                       _const_spec((2 * C, C)), _const_spec((1, C)),
                       _const_spec((1, C)), _const_spec((1, C)),
                       _const_spec((1, C)), _const_spec((1, C))]
        agg_spec = pl.BlockSpec((2, BT, 32), lambda i: (_Z(), i, _Z()))

        if l < NL - 1:
            h, q, kv, ra, rb = pl.pallas_call(
                _mid_body,
                grid=(NMT,),
                in_specs=[row_spec, agg_spec, row_spec] + mid_w_specs +
                         [_const_spec((C, 3 * C)), _const_spec((1, 3 * C)),
                          _const_spec((4, C))],
                out_specs=[row_spec, row_spec, kv_spec] + [r_spec] * 2,
                out_shape=[nodes_f32, nodes_f32, kv_f32] + [rtab] * 2,
            )(h, agg3, h2, *mid_w, wiT[l + 1], biT[l + 1], emb)
        else:
            pooled = pl.pallas_call(
                _pool_body,
                grid=(NMT,),
                in_specs=[row_spec, agg_spec, row_spec] + mid_w_specs +
                         [pl.BlockSpec((1, 1, BT), lambda i: (i, _Z(), _Z()))],
                out_specs=pl.BlockSpec((G, C), lambda i: (_Z(), _Z())),
                out_shape=jax.ShapeDtypeStruct((G, C), F32),
            )(h, agg3, h2, *mid_w, bat3)

    return pooled


# speedup vs baseline: 41.1349x; 1.1377x over previous
import functools

import jax
import jax.numpy as jnp
from jax import lax
from jax.experimental import pallas as pl
from jax.experimental.pallas import tpu as pltpu
from jax.experimental.pallas import tpu_sc as plsc

N = 50000
E = 800000
G = 1000
C = 64
H = 4
DH = 16
NL = 3

TQ = 1000
TK = 400
NQT = N // TQ

BT = 1000
NMT = N // BT

ER = E // 128
ERP = 6256
EGB = 16
NGRP = ERP // EGB

NSUB = 16
NP = 50048
NZR = NP // NSUB
WB = 184
NWB = NZR // WB

NEG = -1e9
F32 = jnp.float32


def _eidx_body(a0, a1, a2, a3, s_ref, gid_ref):
    best = a0[...]
    bi = jnp.zeros(best.shape, jnp.int32)
    for j, ar in enumerate((a1, a2, a3), start=1):
        v = ar[...]
        bi = jnp.where(v > best, j, bi)
        best = jnp.maximum(best, v)
    gid_ref[...] = bi * N + s_ref[...]


def _emit_pre(h, wi_ref, bi_ref, emb_ref, q_ref, kv_ref, ra_ref, rb_ref):
    qkv = jnp.dot(h, wi_ref[...], preferred_element_type=F32) + bi_ref[...]
    q_ref[...] = qkv[:, 0:C]
    kv_ref[...] = qkv[:, C:3 * C]
    for cc in range(4):
        rc = jnp.maximum(h + emb_ref[cc:cc + 1, :], 0.0)
        ra_ref[cc, :, :] = rc[:, :32]
        rb_ref[cc, :, :] = rc[:, 32:]


def _k0_body(x_ref, w0_ref, b0_ref, wi_ref, bi_ref, emb_ref,
             h_ref, q_ref, kv_ref, ra_ref, rb_ref):
    h = jnp.dot(x_ref[...], w0_ref[...], preferred_element_type=F32) + b0_ref[...]
    h_ref[...] = h
    _emit_pre(h, wi_ref, bi_ref, emb_ref, q_ref, kv_ref, ra_ref, rb_ref)


def _mid_common(h_ref, agg_ref, h2_ref, w1, b1, w2, b2, mw1, mb1, mw2, mb2,
                n1g, n1b, n3g, n3b):
    h = h_ref[...]
    ag = agg_ref[...]
    agg = jnp.concatenate([ag[0], ag[1]], axis=1)
    t = h + agg
    t = jnp.dot(jnp.maximum(jnp.dot(t, w1[...], preferred_element_type=F32)
                            + b1[...], 0.0),
                w2[...], preferred_element_type=F32) + b2[...]
    h1 = (t + h) * n1g[...] + n1b[...]
    out = h1 + h2_ref[...]
    mlp = jnp.dot(jnp.maximum(jnp.dot(out, mw1[...], preferred_element_type=F32)
                              + mb1[...], 0.0),
                  mw2[...], preferred_element_type=F32) + mb2[...]
    return (out + mlp) * n3g[...] + n3b[...]


def _mid_body(h_ref, agg_ref, h2_ref, w1, b1, w2, b2, mw1, mb1, mw2, mb2,
              n1g, n1b, n3g, n3b, wi_ref, bi_ref, emb_ref,
              hn_ref, q_ref, kv_ref, ra_ref, rb_ref):
    hn = _mid_common(h_ref, agg_ref, h2_ref, w1, b1, w2, b2,
                     mw1, mb1, mw2, mb2, n1g, n1b, n3g, n3b)
    hn_ref[...] = hn
    _emit_pre(hn, wi_ref, bi_ref, emb_ref, q_ref, kv_ref, ra_ref, rb_ref)


def _pool_body(h_ref, agg_ref, h2_ref, w1, b1, w2, b2, mw1, mb1, mw2, mb2,
               n1g, n1b, n3g, n3b, bat_ref, pool_ref):
    hn = _mid_common(h_ref, agg_ref, h2_ref, w1, b1, w2, b2,
                     mw1, mb1, mw2, mb2, n1g, n1b, n3g, n3b)
    bt = bat_ref[0]
    oh = (lax.broadcasted_iota(jnp.int32, (G, BT), 0) == bt).astype(F32)
    contrib = jnp.dot(oh, hn, preferred_element_type=F32)

    @pl.when(pl.program_id(0) == 0)
    def _init():
        pool_ref[...] = jnp.zeros_like(pool_ref)

    pool_ref[...] = pool_ref[...] + contrib


def _attn_body(ks_ref, nk_ref, q_ref, qlo_ref, qhi_ref, h_ref,
               wo_ref, bo_ref, g_ref, bb_ref, kv_hbm,
               o_ref, kvbuf, sems):
    i = pl.program_id(0)
    start = ks_ref[i]
    nk = nk_ref[i]
    qlo = qlo_ref[0]
    qhi = qhi_ref[0]

    def desc(kt, par):
        off = start + kt * TK
        return pltpu.make_async_copy(kv_hbm.at[pl.ds(off, TK)],
                                     kvbuf.at[par], sems.at[par])

    desc(jnp.int32(0), jnp.int32(0)).start()

    def body(kt, carry):
        ms, ls, accs = carry
        par = lax.rem(kt, jnp.int32(2))

        @pl.when(kt + 1 < nk)
        def _pf():
            desc(kt + 1, 1 - par).start()

        desc(kt, par).wait()
        kv = kvbuf[par]
        off = start + kt * TK
        col = off + lax.broadcasted_iota(jnp.int32, (TK, TQ), 0)
        madd = jnp.where((col >= qlo) & (col < qhi),
                         jnp.float32(0.0), jnp.float32(NEG))
        nms, nls, naccs = [], [], []
        for hh in range(H):
            qs = q_ref[:, DH * hh:DH * (hh + 1)] * 0.25
            st = lax.dot_general(kv[:, DH * hh:DH * (hh + 1)], qs,
                                 (((1,), (1,)), ((), ())),
                                 preferred_element_type=F32)
            st = st + madd
            mn = jnp.maximum(ms[hh], jnp.max(st, axis=0, keepdims=True))
            al = jnp.exp(ms[hh] - mn)
            pt = jnp.exp(st - mn)
            nls.append(ls[hh] * al + jnp.sum(pt, axis=0, keepdims=True))
            pv = lax.dot_general(kv[:, C + DH * hh:C + DH * (hh + 1)], pt,
                                 (((0,), (0,)), ((), ())),
                                 preferred_element_type=F32)
            naccs.append(accs[hh] * al + pv)
            nms.append(mn)
        return tuple(nms), tuple(nls), tuple(naccs)

    m0 = tuple(jnp.full((1, TQ), -1e30, F32) for _ in range(H))
    l0 = tuple(jnp.zeros((1, TQ), F32) for _ in range(H))
    a0 = tuple(jnp.zeros((DH, TQ), F32) for _ in range(H))
    ms, ls, accs = lax.fori_loop(jnp.int32(0), nk, body, (m0, l0, a0))
    ot = jnp.concatenate([accs[hh] / ls[hh] for hh in range(H)], axis=0)
    a_out = lax.dot_general(ot, wo_ref[...], (((0,), (0,)), ((), ())),
                            preferred_element_type=F32) + bo_ref[...]
    o_ref[...] = (a_out + h_ref[...]) * g_ref[...] + bb_ref[...]


_sc_mesh = plsc.VectorSubcoreMesh(core_axis_name="c", subcore_axis_name="s")


@functools.partial(
    pl.kernel,
    mesh=_sc_mesh,
    compiler_params=pltpu.CompilerParams(use_tc_tiling_on_sc=False),
    out_type=jax.ShapeDtypeStruct((2, NP, 32), F32),
    scratch_types=[
        pltpu.VMEM((EGB, 128), jnp.int32),
        pltpu.VMEM((EGB, 128), jnp.int32),
        pltpu.VMEM((2, 2, 128, 32), F32),
        pltpu.VMEM((WB, 32), F32),
        pltpu.VMEM_SHARED((NP, 32), F32),
        pltpu.SemaphoreType.DMA,
        pltpu.SemaphoreType.DMA,
    ],
)
def _sc_edge_agg(gid_hbm, dst_hbm, r0_hbm, r1_hbm, zero_hbm, out_hbm,
                 gbuf, dbuf, rows, zbuf, aggsh, gsem, ssem):
    c = lax.axis_index("c")
    s = lax.axis_index("s")

    pltpu.sync_copy(zero_hbm, zbuf)
    base = s * NZR
    for j in range(NWB):
        pltpu.sync_copy(zbuf, aggsh.at[pl.ds(base + j * WB, WB)])
    plsc.subcore_barrier()

    ngrp = jnp.int32(NGRP // NSUB) + (s < NGRP % NSUB).astype(jnp.int32)
    nsub = EGB // 2

    def body(i, carry):
        g = s + i * NSUB
        pltpu.sync_copy(gid_hbm.at[pl.ds(g * EGB, EGB)], gbuf)
        pltpu.sync_copy(dst_hbm.at[pl.ds(g * EGB, EGB)], dbuf)

        def pipe(r_hbm):
            def gfire(t):
                p = t % 2
                return [pltpu.async_copy(
                    r_hbm.at[gbuf.at[jnp.int32(2 * t + j)]],
                    rows.at[jnp.int32(p), jnp.int32(j)], gsem)
                    for j in range(2)]

            def sfire(t):
                p = t % 2
                return [pltpu.async_copy(
                    rows.at[jnp.int32(p), jnp.int32(j)],
                    aggsh.at[dbuf.at[jnp.int32(2 * t + j)]], ssem, add=True)
                    for j in range(2)]

            gd = gfire(0)
            sd_prev = None
            for t in range(nsub):
                for d in gd:
                    d.wait()
                sd = sfire(t)
                if sd_prev is not None:
                    for d in sd_prev:
                        d.wait()
                if t < nsub - 1:
                    gd = gfire(t + 1)
                sd_prev = sd
            for d in sd_prev:
                d.wait()

        @pl.when(c == 0)
        def _p0():
            pipe(r0_hbm)

        @pl.when(c == 1)
        def _p1():
            pipe(r1_hbm)

        return carry

    lax.fori_loop(jnp.int32(0), ngrp, body, jnp.int32(0))
    plsc.subcore_barrier()

    for j in range(NWB):
        pltpu.sync_copy(aggsh.at[pl.ds(base + j * WB, WB)], zbuf)
        pltpu.sync_copy(zbuf, out_hbm.at[c, pl.ds(base + j * WB, WB)])


def _Z():
    return jnp.int32(0)


def _const_spec(shape):
    return pl.BlockSpec(shape, lambda i: tuple(_Z() for _ in shape))


def kernel(x, pe, edge_attr, params, edge_index, batch):
    p = params
    batch32 = batch.astype(jnp.int32)
    src = edge_index[0].astype(jnp.int32)
    dst = edge_index[1].astype(jnp.int32)

    xf = x.astype(F32)
    X = jnp.concatenate([xf, pe.astype(F32), jnp.zeros((N, 3), F32)], axis=1)
    w0 = jnp.zeros((32, C), F32)
    w0 = w0.at[0:9, 0:C - 8].set(p["node_W"].T)
    pe_w = p["pe_W"] * p["pe_norm_g"][None, :]
    pe_b = p["pe_W"] @ p["pe_norm_b"] + p["pe_b"]
    w0 = w0.at[9:29, C - 8:C].set(pe_w.T)
    b0 = jnp.concatenate([p["node_b"], pe_b])[None, :]

    emb = p["edge_emb"].astype(F32)
    wiT = [p["attn_Wi"][l].T for l in range(NL)]
    biT = [p["attn_bi"][l][None, :] for l in range(NL)]
    woT = [p["attn_Wo"][l].T for l in range(NL)]
    boT = [p["attn_bo"][l][None, :] for l in range(NL)]
    w1T = [p["nn_W1"][l].T for l in range(NL)]
    b1T = [p["nn_b1"][l][None, :] for l in range(NL)]
    w2T = [p["nn_W2"][l].T for l in range(NL)]
    b2T = [p["nn_b2"][l][None, :] for l in range(NL)]
    mw1T = [p["mlp_W1"][l].T for l in range(NL)]
    mb1T = [p["mlp_b1"][l][None, :] for l in range(NL)]
    mw2T = [p["mlp_W2"][l].T for l in range(NL)]
    mb2T = [p["mlp_b2"][l][None, :] for l in range(NL)]
    ng = {nm: [p[nm + "_g"][l][None, :] for l in range(NL)] for nm in ("n1", "n2", "n3")}
    nb = {nm: [p[nm + "_b"][l][None, :] for l in range(NL)] for nm in ("n1", "n2", "n3")}

    pad_e = ERP * 128 - E

    def _erp(v, dt, fill=0):
        return jnp.pad(v.astype(dt), (0, pad_e),
                       constant_values=fill).reshape(ERP, 128)

    acols = [_erp(edge_attr[:, j], F32) for j in range(4)]
    src2 = _erp(src, jnp.int32)
    dst2 = _erp(dst, jnp.int32, fill=N)

    ar = jnp.arange(N, dtype=jnp.int32)
    starts = jnp.concatenate([jnp.ones((1,), jnp.bool_),
                              batch32[1:] != batch32[:-1]])
    ends = jnp.concatenate([batch32[:-1] != batch32[1:],
                            jnp.ones((1,), jnp.bool_)])
    row_lo = lax.cummax(jnp.where(starts, ar, 0))
    row_hi = lax.cummin(jnp.where(ends, ar + 1, N), reverse=True)
    qlo3 = row_lo.reshape(NQT, 1, TQ)
    qhi3 = row_hi.reshape(NQT, 1, TQ)
    kstart = (row_lo[::TQ] // TK) * TK
    nkt = (row_hi[TQ - 1::TQ] - kstart + TK - 1) // TK
    bat3 = batch32.reshape(NMT, 1, BT)
    zero_wb = jnp.zeros((WB, 32), F32)

    gid2 = pl.pallas_call(
        _eidx_body,
        out_shape=jax.ShapeDtypeStruct((ERP, 128), jnp.int32),
    )(*acols, src2)

    row_spec = pl.BlockSpec((BT, C), lambda i: (i, _Z()))
    kv_spec = pl.BlockSpec((BT, 2 * C), lambda i: (i, _Z()))
    r_spec = pl.BlockSpec((4, BT, 32), lambda i: (_Z(), i, _Z()))
    nodes_f32 = jax.ShapeDtypeStruct((N, C), F32)
    kv_f32 = jax.ShapeDtypeStruct((N, 2 * C), F32)
    rtab = jax.ShapeDtypeStruct((4, N, 32), F32)

    h, q, kv, ra, rb = pl.pallas_call(
        _k0_body,
        grid=(NMT,),
        in_specs=[pl.BlockSpec((BT, 32), lambda i: (i, _Z())),
                  _const_spec((32, C)), _const_spec((1, C)),
                  _const_spec((C, 3 * C)), _const_spec((1, 3 * C)),
                  _const_spec((4, C))],
        out_specs=[row_spec, row_spec, kv_spec] + [r_spec] * 2,
        out_shape=[nodes_f32, nodes_f32, kv_f32] + [rtab] * 2,
    )(X, w0, b0, wiT[0], biT[0], emb)

    pooled = None
    for l in range(NL):
        agg3 = _sc_edge_agg(gid2, dst2, ra.reshape(4 * N, 32),
                            rb.reshape(4 * N, 32), zero_wb)

        h2 = pl.pallas_call(
            _attn_body,
            grid=(NQT,),
            in_specs=[pl.BlockSpec((NQT,), lambda i: (_Z(),),
                                   memory_space=pltpu.SMEM),
                      pl.BlockSpec((NQT,), lambda i: (_Z(),),
                                   memory_space=pltpu.SMEM),
                      pl.BlockSpec((TQ, C), lambda i: (i, _Z())),
                      pl.BlockSpec((1, 1, TQ), lambda i: (i, _Z(), _Z())),
                      pl.BlockSpec((1, 1, TQ), lambda i: (i, _Z(), _Z())),
                      pl.BlockSpec((TQ, C), lambda i: (i, _Z())),
                      _const_spec((C, C)), _const_spec((1, C)),
                      _const_spec((1, C)), _const_spec((1, C)),
                      pl.BlockSpec(memory_space=pl.ANY)],
            out_specs=pl.BlockSpec((TQ, C), lambda i: (i, _Z())),
            out_shape=nodes_f32,
            scratch_shapes=[pltpu.VMEM((2, TK, 2 * C), F32),
                            pltpu.SemaphoreType.DMA((2,))],
        )(kstart, nkt, q, qlo3, qhi3, h, woT[l], boT[l],
          ng["n2"][l], nb["n2"][l], kv)

        mid_w = (w1T[l], b1T[l], w2T[l], b2T[l],
                 mw1T[l], mb1T[l], mw2T[l], mb2T[l],
                 ng["n1"][l], nb["n1"][l], ng["n3"][l], nb["n3"][l])
        mid_w_specs = [_const_spec((C, C)), _const_spec((1, C)),
                       _const_spec((C, C)), _const_spec((1, C)),
                       _const_spec((C, 2 * C)), _const_spec((1, 2 * C)),
                       _const_spec((2 * C, C)), _const_spec((1, C)),
                       _const_spec((1, C)), _const_spec((1, C)),
                       _const_spec((1, C)), _const_spec((1, C))]
        agg_spec = pl.BlockSpec((2, BT, 32), lambda i: (_Z(), i, _Z()))

        if l < NL - 1:
            h, q, kv, ra, rb = pl.pallas_call(
                _mid_body,
                grid=(NMT,),
                in_specs=[row_spec, agg_spec, row_spec] + mid_w_specs +
                         [_const_spec((C, 3 * C)), _const_spec((1, 3 * C)),
                          _const_spec((4, C))],
                out_specs=[row_spec, row_spec, kv_spec] + [r_spec] * 2,
                out_shape=[nodes_f32, nodes_f32, kv_f32] + [rtab] * 2,
            )(h, agg3, h2, *mid_w, wiT[l + 1], biT[l + 1], emb)
        else:
            pooled = pl.pallas_call(
                _pool_body,
                grid=(NMT,),
                in_specs=[row_spec, agg_spec, row_spec] + mid_w_specs +
                         [pl.BlockSpec((1, 1, BT), lambda i: (i, _Z(), _Z()))],
                out_specs=pl.BlockSpec((G, C), lambda i: (_Z(), _Z())),
                out_shape=jax.ShapeDtypeStruct((G, C), F32),
            )(h, agg3, h2, *mid_w, bat3)

    return pooled
